# Initial kernel scaffold; baseline (speedup 1.0000x reference)
import jax
import jax.numpy as jnp
from jax.experimental import pallas as pl


def kernel(edge_vector, edge_idx, atom_type, atom_feats_0, radial_W, mlp0_W0, mlp0_b0, mlp0_W1, mlp0_b1, mlp0_W2, mlp0_b2, mlp1_W0, mlp1_b0, mlp1_W1, mlp1_b1, mlp1_W2, mlp1_b2, mlp2_W0, mlp2_b0, mlp2_W1, mlp2_b1, mlp2_W2, mlp2_b2, chan0_W, chan1_W, chan2_W):
    raise NotImplementedError("write your pallas kernel here")


# speedup vs baseline: 84.6319x; 84.6319x over previous
import functools

import jax
import jax.numpy as jnp
from jax import lax
from jax.experimental import pallas as pl
from jax.experimental.pallas import tpu as pltpu
from jax.experimental.pallas import tpu_sc as plsc

N_NODES = 10000
N_EDGES = 320000
N_U = 8
N_CHEB = 9
R_CUT = 5.0
NUM_AVG_NEIGH = 32.0

NW = 32
EPT = N_EDGES // NW
CH = 80
NCHUNK = EPT // CH
PCH = 2000
MROW = 112
BLK = 1280
NPTA = 624
NPTL = N_NODES - 15 * NPTA


def _gather_body(i_hbm, j_hbm, type_hbm, feats_hbm, h0j_hbm, pair8_hbm,
                 type_v, iflat, jflat, hbuf, pbuf, sem):
    cid = lax.axis_index("c")
    sid = lax.axis_index("s")
    wid = sid * 2 + cid
    base = wid * EPT
    pltpu.sync_copy(type_hbm, type_v)
    pltpu.sync_copy(i_hbm.at[pl.ds(base, EPT)], iflat)
    pltpu.sync_copy(j_hbm.at[pl.ds(base, EPT)], jflat)

    def pair_chunk(c5, _):
        def grp(g, _):
            off = c5 * PCH + g * 16
            iv = iflat[pl.ds(off, 16)]
            jv = jflat[pl.ds(off, 16)]
            it = plsc.load_gather(type_v, [iv])
            jt = plsc.load_gather(type_v, [jv])
            pv = (it * 4 + jt).astype(jnp.float32)
            for u in range(8):
                pbuf[pl.ds(u * PCH + g * 16, 16)] = pv
            return 0
        lax.fori_loop(0, PCH // 16, grp, 0)
        for u in range(8):
            pltpu.sync_copy(pbuf.at[pl.ds(u * PCH, PCH)],
                            pair8_hbm.at[pl.ds(u * N_EDGES + base + c5 * PCH, PCH)])
        return 0
    lax.fori_loop(0, EPT // PCH, pair_chunk, 0)

    def h_chunk(c, _):
        pltpu.async_copy(feats_hbm.at[jflat.at[pl.ds(c * CH, CH)]], hbuf, sem).wait()
        pltpu.sync_copy(hbuf, h0j_hbm.at[pl.ds(base + c * CH, CH)])
        return 0
    lax.fori_loop(0, NCHUNK, h_chunk, 0)


@functools.partial(jax.jit, static_argnames=())
def _sc_gather(i_row, j_row, atom_type, featsN8):
    mesh = plsc.VectorSubcoreMesh(core_axis_name="c", subcore_axis_name="s")
    fn = pl.kernel(
        _gather_body,
        out_type=(jax.ShapeDtypeStruct((N_EDGES, 8), jnp.float32),
                  jax.ShapeDtypeStruct((8 * N_EDGES,), jnp.float32)),
        mesh=mesh,
        scratch_types=[
            pltpu.VMEM((N_NODES,), jnp.int32),
            pltpu.VMEM((EPT,), jnp.int32),
            pltpu.VMEM((EPT,), jnp.int32),
            pltpu.VMEM((CH, 8), jnp.float32),
            pltpu.VMEM((8 * PCH,), jnp.float32),
            pltpu.SemaphoreType.DMA,
        ],
        compiler_params=pltpu.CompilerParams(needs_layout_passes=False, use_tc_tiling_on_sc=False),
    )
    return fn(i_row, j_row, atom_type, featsN8)


def _dense_body(evp, pairb, h0jb, tab, w0, w1, w2, b0, b1, b2, out):
    x = evp[0:1, :]
    y = evp[1:2, :]
    z = evp[2:3, :]
    r = jnp.sqrt(x * x + y * y + z * z)
    xc = jnp.clip(2.0 * r / R_CUT - 1.0, -1.0, 1.0)
    Ts = [jnp.ones_like(xc), xc]
    for _ in range(2, N_CHEB):
        Ts.append(2.0 * xc * Ts[-1] - Ts[-2])
    Tcm = jnp.concatenate(Ts, axis=0)
    xr = r * (1.0 / R_CUT)
    x2 = xr * xr
    x3 = x2 * xr
    x6 = x3 * x3
    env = jnp.where(xr < 1.0,
                    1.0 - 28.0 * x6 + 48.0 * x6 * xr - 21.0 * x6 * x2,
                    0.0)
    Gm = jnp.dot(tab[...], Tcm, preferred_element_type=jnp.float32)
    pair = pairb[...]
    fu8 = jnp.zeros((8, BLK), jnp.float32)
    for p in range(16):
        fu8 = fu8 + jnp.where(pair == float(p), Gm[8 * p:8 * p + 8, :], 0.0)
    fu = env * fu8
    h1 = jnp.dot(w0[...], fu, preferred_element_type=jnp.float32) + b0[...]
    h1 = h1 * (1.0 / (1.0 + jnp.exp(-h1)))
    h2 = jnp.dot(w1[...], h1, preferred_element_type=jnp.float32) + b1[...]
    h2 = h2 * (1.0 / (1.0 + jnp.exp(-h2)))
    Rm = jnp.dot(w2[...], h2, preferred_element_type=jnp.float32) + b2[...]
    h0T = h0jb[...].T
    Rh = Rm * jnp.concatenate([h0T, h0T, h0T], axis=0)
    rs = jnp.maximum(r, 1e-12)
    unit = evp[0:3, :] / rs
    uu = jnp.concatenate([unit * unit[k:k + 1, :] for k in range(3)], axis=0)
    pieces = [Rh[0:8, :]]
    for a in range(3):
        pieces.append(Rh[8:16, :] * unit[a:a + 1, :])
    for c in range(9):
        pieces.append(Rh[16:24, :] * uu[c:c + 1, :])
    pieces.append(jnp.zeros((8, BLK), jnp.float32))
    out[...] = jnp.concatenate(pieces, axis=0).T


def _tc_dense(evp, pair8, h0j, tab, w0, w1, w2, b0, b1, b2):
    nblk = N_EDGES // BLK
    return pl.pallas_call(
        _dense_body,
        grid=(nblk,),
        in_specs=[
            pl.BlockSpec((8, BLK), lambda i: (0, i)),
            pl.BlockSpec((8, BLK), lambda i: (0, i)),
            pl.BlockSpec((BLK, 8), lambda i: (i, 0)),
            pl.BlockSpec((128, N_CHEB), lambda i: (0, 0)),
            pl.BlockSpec((24, 8), lambda i: (0, 0)),
            pl.BlockSpec((24, 24), lambda i: (0, 0)),
            pl.BlockSpec((24, 24), lambda i: (0, 0)),
            pl.BlockSpec((24, 1), lambda i: (0, 0)),
            pl.BlockSpec((24, 1), lambda i: (0, 0)),
            pl.BlockSpec((24, 1), lambda i: (0, 0)),
        ],
        out_specs=pl.BlockSpec((BLK, MROW), lambda i: (i, 0)),
        out_shape=jax.ShapeDtypeStruct((N_EDGES, MROW), jnp.float32),
    )(evp, pair8, h0j, tab, w0, w1, w2, b0, b1, b2)


def _scatter_body(msg_hbm, i2d_hbm, zrows_hbm, part_hbm, ivm, mbuf, acc, sem):
    cid = lax.axis_index("c")
    sid = lax.axis_index("s")
    wid = sid * 2 + cid
    pltpu.sync_copy(i2d_hbm.at[wid], ivm)

    @pl.when(sid < 15)
    def _():
        pltpu.sync_copy(zrows_hbm.at[pl.ds(sid * NPTA, NPTA)],
                        acc.at[pl.ds(sid * NPTA, NPTA)])

    @pl.when(sid == 15)
    def _():
        pltpu.sync_copy(zrows_hbm.at[pl.ds(15 * NPTA, NPTL)],
                        acc.at[pl.ds(15 * NPTA, NPTL)])
    plsc.subcore_barrier()

    def body(c, _):
        pltpu.sync_copy(msg_hbm.at[pl.ds(wid * EPT + c * CH, CH)], mbuf)
        pltpu.sync_copy(mbuf, acc.at[ivm.at[c]], add=True)
        return 0
    lax.fori_loop(0, NCHUNK, body, 0)
    plsc.subcore_barrier()

    @pl.when(sid < 15)
    def _():
        pltpu.sync_copy(acc.at[pl.ds(sid * NPTA, NPTA)],
                        part_hbm.at[cid, pl.ds(sid * NPTA, NPTA)])

    @pl.when(sid == 15)
    def _():
        pltpu.sync_copy(acc.at[pl.ds(15 * NPTA, NPTL)],
                        part_hbm.at[cid, pl.ds(15 * NPTA, NPTL)])


def _sc_scatter(msg, i2d, zrows):
    mesh = plsc.VectorSubcoreMesh(core_axis_name="c", subcore_axis_name="s")
    fn = pl.kernel(
        _scatter_body,
        out_type=jax.ShapeDtypeStruct((2, N_NODES, MROW), jnp.float32),
        mesh=mesh,
        scratch_types=[
            pltpu.VMEM((NCHUNK, CH), jnp.int32),
            pltpu.VMEM((CH, MROW), jnp.float32),
            pltpu.VMEM_SHARED((N_NODES, MROW), jnp.float32),
            pltpu.SemaphoreType.DMA,
        ],
        compiler_params=pltpu.CompilerParams(needs_layout_passes=False, use_tc_tiling_on_sc=False),
    )
    return fn(msg, i2d, zrows)


def _finish_body(part, wb, out):
    s = (part[0] + part[1]) * (1.0 / (NUM_AVG_NEIGH ** 0.5))
    out[...] = jnp.dot(s, wb[...], preferred_element_type=jnp.float32)


def _tc_finish(part, wb):
    return pl.pallas_call(
        _finish_body,
        out_shape=jax.ShapeDtypeStruct((N_NODES, MROW), jnp.float32),
    )(part, wb)


def kernel(edge_vector, edge_idx, atom_type, atom_feats_0, radial_W,
           mlp0_W0, mlp0_b0, mlp0_W1, mlp0_b1, mlp0_W2, mlp0_b2,
           mlp1_W0, mlp1_b0, mlp1_W1, mlp1_b1, mlp1_W2, mlp1_b2,
           mlp2_W0, mlp2_b0, mlp2_W1, mlp2_b1, mlp2_W2, mlp2_b2,
           chan0_W, chan1_W, chan2_W):
    f32 = jnp.float32
    i_row = edge_idx[0]
    j_row = edge_idx[1]
    featsN8 = atom_feats_0.T

    tab = radial_W.reshape(16, N_U, N_CHEB).reshape(128, N_CHEB)
    w0 = jnp.concatenate([mlp0_W0, mlp1_W0, mlp2_W0], axis=1).T

    def bd(a, b, c):
        z = jnp.zeros((24, 24), f32)
        return z.at[0:8, 0:8].set(a).at[8:16, 8:16].set(b).at[16:24, 16:24].set(c)

    w1 = bd(mlp0_W1, mlp1_W1, mlp2_W1).T
    w2 = bd(mlp0_W2, mlp1_W2, mlp2_W2).T
    b0 = jnp.concatenate([mlp0_b0, mlp1_b0, mlp2_b0])[:, None]
    b1 = jnp.concatenate([mlp0_b1, mlp1_b1, mlp2_b1])[:, None]
    b2 = jnp.concatenate([mlp0_b2, mlp1_b2, mlp2_b2])[:, None]
    wb = jnp.zeros((MROW, MROW), f32)
    wb = wb.at[0:8, 0:8].set(chan0_W.T)
    wb = wb.at[8:32, 8:32].set(jnp.kron(jnp.eye(3, dtype=f32), chan1_W.T))
    wb = wb.at[32:104, 32:104].set(jnp.kron(jnp.eye(9, dtype=f32), chan2_W.T))

    evp = jnp.concatenate([edge_vector.T,
                           jnp.zeros((5, N_EDGES), f32)], axis=0)

    h0j, pair8f = _sc_gather(i_row, j_row, atom_type, featsN8)
    pair8 = pair8f.reshape(8, N_EDGES)
    msg = _tc_dense(evp, pair8, h0j, tab, w0, w1, w2, b0, b1, b2)
    i2d = i_row.reshape(NW, NCHUNK, CH)
    zrows = jnp.zeros((N_NODES, MROW), f32)
    part = _sc_scatter(msg, i2d, zrows)
    out = _tc_finish(part, wb)

    M0 = out[:, 0:8].T
    M1 = out[:, 8:32].reshape(N_NODES, 3, 8).transpose(2, 0, 1)
    M2 = out[:, 32:104].reshape(N_NODES, 3, 3, 8).transpose(3, 0, 2, 1)
    return (M0, M1, M2)


# speedup vs baseline: 163.1103x; 1.9273x over previous
import functools

import jax
import jax.numpy as jnp
from jax import lax
from jax.experimental import pallas as pl
from jax.experimental.pallas import tpu as pltpu
from jax.experimental.pallas import tpu_sc as plsc

N_NODES = 10000
N_EDGES = 320000
N_U = 8
N_CHEB = 9
R_CUT = 5.0
NUM_AVG_NEIGH = 32.0

NW = 32
EPT = N_EDGES // NW
CH = 80
NCHUNK = EPT // CH
PCH = 2000
MROW = 128
BLK = 1280
NPTA = 624
NPTL = N_NODES - 15 * NPTA

_SC_PARAMS = pltpu.CompilerParams(needs_layout_passes=False,
                                  use_tc_tiling_on_sc=False)


def _gather_body(i_hbm, j_hbm, type_hbm, ftab_hbm, h0T_hbm, pair8_hbm,
                 type_v, ftab, ibuf, jbuf, sbuf, sem):
    cid = lax.axis_index("c")
    sid = lax.axis_index("s")
    wid = sid * 2 + cid
    base = wid * EPT
    pltpu.sync_copy(type_hbm, type_v)
    pltpu.sync_copy(ftab_hbm, ftab)

    def chunk(c5, _):
        cb = base + c5 * PCH
        pltpu.sync_copy(i_hbm.at[pl.ds(cb, PCH)], ibuf)
        pltpu.sync_copy(j_hbm.at[pl.ds(cb, PCH)], jbuf)

        def grp_pair(g, _):
            iv = ibuf[pl.ds(g * 16, 16)]
            jv = jbuf[pl.ds(g * 16, 16)]
            it = plsc.load_gather(type_v, [iv])
            jt = plsc.load_gather(type_v, [jv])
            pv = (it * 4 + jt).astype(jnp.float32)
            for u in range(8):
                sbuf[u, pl.ds(g * 16, 16)] = pv
            return 0
        lax.fori_loop(0, PCH // 16, grp_pair, 0)
        pltpu.sync_copy(sbuf, pair8_hbm.at[:, pl.ds(cb, PCH)])

        def grp_h0(g, _):
            jv = jbuf[pl.ds(g * 16, 16)]
            j8 = jv * 8
            for u in range(8):
                hv = plsc.load_gather(ftab, [j8 + u])
                sbuf[u, pl.ds(g * 16, 16)] = hv
            return 0
        lax.fori_loop(0, PCH // 16, grp_h0, 0)
        pltpu.sync_copy(sbuf, h0T_hbm.at[:, pl.ds(cb, PCH)])
        return 0
    lax.fori_loop(0, EPT // PCH, chunk, 0)


@jax.jit
def _sc_gather(i_row, j_row, atom_type, ftab_flat):
    mesh = plsc.VectorSubcoreMesh(core_axis_name="c", subcore_axis_name="s")
    fn = pl.kernel(
        _gather_body,
        out_type=(jax.ShapeDtypeStruct((8, N_EDGES), jnp.float32),
                  jax.ShapeDtypeStruct((8, N_EDGES), jnp.float32)),
        mesh=mesh,
        scratch_types=[
            pltpu.VMEM((N_NODES,), jnp.int32),
            pltpu.VMEM((N_NODES * 8,), jnp.float32),
            pltpu.VMEM((PCH,), jnp.int32),
            pltpu.VMEM((PCH,), jnp.int32),
            pltpu.VMEM((8, PCH), jnp.float32),
            pltpu.SemaphoreType.DMA,
        ],
        compiler_params=_SC_PARAMS,
    )
    return fn(i_row, j_row, atom_type, ftab_flat)


def _dense_body(evp, pairb, h0Tb, tab, w0, w1, w2, b0, b1, b2, out):
    x = evp[0:1, :]
    y = evp[1:2, :]
    z = evp[2:3, :]
    r = jnp.sqrt(x * x + y * y + z * z)
    xc = jnp.clip(2.0 * r / R_CUT - 1.0, -1.0, 1.0)
    Ts = [jnp.ones_like(xc), xc]
    for _ in range(2, N_CHEB):
        Ts.append(2.0 * xc * Ts[-1] - Ts[-2])
    Tcm = jnp.concatenate(Ts, axis=0)
    xr = r * (1.0 / R_CUT)
    x2 = xr * xr
    x3 = x2 * xr
    x6 = x3 * x3
    env = jnp.where(xr < 1.0,
                    1.0 - 28.0 * x6 + 48.0 * x6 * xr - 21.0 * x6 * x2,
                    0.0)
    Gm = jnp.dot(tab[...], Tcm, preferred_element_type=jnp.float32)
    pair = pairb[...]
    fu8 = jnp.zeros((8, BLK), jnp.float32)
    for p in range(16):
        fu8 = fu8 + jnp.where(pair == float(p), Gm[8 * p:8 * p + 8, :], 0.0)
    fu = env * fu8
    h1 = jnp.dot(w0[...], fu, preferred_element_type=jnp.float32) + b0[...]
    h1 = h1 * (1.0 / (1.0 + jnp.exp(-h1)))
    h2 = jnp.dot(w1[...], h1, preferred_element_type=jnp.float32) + b1[...]
    h2 = h2 * (1.0 / (1.0 + jnp.exp(-h2)))
    Rm = jnp.dot(w2[...], h2, preferred_element_type=jnp.float32) + b2[...]
    h0T = h0Tb[...]
    Rh = Rm * jnp.concatenate([h0T, h0T, h0T], axis=0)
    rs = jnp.maximum(r, 1e-12)
    unit = evp[0:3, :] / rs
    uu = jnp.concatenate([unit * unit[k:k + 1, :] for k in range(3)], axis=0)
    pieces = [Rh[0:8, :]]
    for a in range(3):
        pieces.append(Rh[8:16, :] * unit[a:a + 1, :])
    for c in range(9):
        pieces.append(Rh[16:24, :] * uu[c:c + 1, :])
    pieces.append(jnp.zeros((MROW - 104, BLK), jnp.float32))
    out[...] = jnp.concatenate(pieces, axis=0).T


def _tc_dense(evp, pair8, h0T, tab, w0, w1, w2, b0, b1, b2):
    nblk = N_EDGES // BLK
    return pl.pallas_call(
        _dense_body,
        grid=(nblk,),
        in_specs=[
            pl.BlockSpec((8, BLK), lambda i: (0, i)),
            pl.BlockSpec((8, BLK), lambda i: (0, i)),
            pl.BlockSpec((8, BLK), lambda i: (0, i)),
            pl.BlockSpec((128, N_CHEB), lambda i: (0, 0)),
            pl.BlockSpec((24, 8), lambda i: (0, 0)),
            pl.BlockSpec((24, 24), lambda i: (0, 0)),
            pl.BlockSpec((24, 24), lambda i: (0, 0)),
            pl.BlockSpec((24, 1), lambda i: (0, 0)),
            pl.BlockSpec((24, 1), lambda i: (0, 0)),
            pl.BlockSpec((24, 1), lambda i: (0, 0)),
        ],
        out_specs=pl.BlockSpec((BLK, MROW), lambda i: (i, 0)),
        out_shape=jax.ShapeDtypeStruct((N_EDGES, MROW), jnp.float32),
    )(evp, pair8, h0T, tab, w0, w1, w2, b0, b1, b2)


def _scatter_body(msg_hbm, i2d_hbm, zrows_hbm, part_hbm, ivm, mb0, mb1, acc,
                  sem0, sem1):
    cid = lax.axis_index("c")
    sid = lax.axis_index("s")
    wid = sid * 2 + cid
    ebase = wid * EPT
    pltpu.sync_copy(i2d_hbm.at[wid], ivm)

    @pl.when(sid < 15)
    def _():
        pltpu.sync_copy(zrows_hbm.at[pl.ds(sid * NPTA, NPTA)],
                        acc.at[pl.ds(sid * NPTA, NPTA)])

    @pl.when(sid == 15)
    def _():
        pltpu.sync_copy(zrows_hbm.at[pl.ds(15 * NPTA, NPTL)],
                        acc.at[pl.ds(15 * NPTA, NPTL)])
    plsc.subcore_barrier()

    def rows(c):
        return pl.ds(ebase + c * CH, CH)

    pltpu.async_copy(msg_hbm.at[rows(0)], mb0, sem0)

    def pairbody(p, _):
        c0 = 2 * p
        c1 = 2 * p + 1
        pltpu.async_copy(msg_hbm.at[rows(c1)], mb1, sem1)
        pltpu.make_async_copy(msg_hbm.at[rows(c0)], mb0, sem0).wait()
        pltpu.sync_copy(mb0, acc.at[ivm.at[c0]], add=True)
        pltpu.async_copy(msg_hbm.at[rows(c0 + 2)], mb0, sem0)
        pltpu.make_async_copy(msg_hbm.at[rows(c1)], mb1, sem1).wait()
        pltpu.sync_copy(mb1, acc.at[ivm.at[c1]], add=True)
        return 0
    lax.fori_loop(0, (NCHUNK - 1) // 2, pairbody, 0)
    pltpu.make_async_copy(msg_hbm.at[rows(NCHUNK - 1)], mb0, sem0).wait()
    pltpu.sync_copy(mb0, acc.at[ivm.at[NCHUNK - 1]], add=True)
    plsc.subcore_barrier()

    @pl.when(sid < 15)
    def _():
        pltpu.sync_copy(acc.at[pl.ds(sid * NPTA, NPTA)],
                        part_hbm.at[cid, pl.ds(sid * NPTA, NPTA)])

    @pl.when(sid == 15)
    def _():
        pltpu.sync_copy(acc.at[pl.ds(15 * NPTA, NPTL)],
                        part_hbm.at[cid, pl.ds(15 * NPTA, NPTL)])


def _sc_scatter(msg, i2d, zrows):
    mesh = plsc.VectorSubcoreMesh(core_axis_name="c", subcore_axis_name="s")
    fn = pl.kernel(
        _scatter_body,
        out_type=jax.ShapeDtypeStruct((2, N_NODES, MROW), jnp.float32),
        mesh=mesh,
        scratch_types=[
            pltpu.VMEM((NCHUNK, CH), jnp.int32),
            pltpu.VMEM((CH, MROW), jnp.float32),
            pltpu.VMEM((CH, MROW), jnp.float32),
            pltpu.VMEM_SHARED((N_NODES, MROW), jnp.float32),
            pltpu.SemaphoreType.DMA,
            pltpu.SemaphoreType.DMA,
        ],
        compiler_params=_SC_PARAMS,
    )
    return fn(msg, i2d, zrows)


def _finish_body(part, wb, out):
    s = (part[0] + part[1]) * (1.0 / (NUM_AVG_NEIGH ** 0.5))
    out[...] = jnp.dot(s, wb[...], preferred_element_type=jnp.float32)


def _tc_finish(part, wb):
    return pl.pallas_call(
        _finish_body,
        out_shape=jax.ShapeDtypeStruct((N_NODES, MROW), jnp.float32),
    )(part, wb)


def kernel(edge_vector, edge_idx, atom_type, atom_feats_0, radial_W,
           mlp0_W0, mlp0_b0, mlp0_W1, mlp0_b1, mlp0_W2, mlp0_b2,
           mlp1_W0, mlp1_b0, mlp1_W1, mlp1_b1, mlp1_W2, mlp1_b2,
           mlp2_W0, mlp2_b0, mlp2_W1, mlp2_b1, mlp2_W2, mlp2_b2,
           chan0_W, chan1_W, chan2_W):
    f32 = jnp.float32
    i_row = edge_idx[0]
    j_row = edge_idx[1]
    ftab_flat = atom_feats_0.T.reshape(N_NODES * 8)

    tab = radial_W.reshape(16, N_U, N_CHEB).reshape(128, N_CHEB)
    w0 = jnp.concatenate([mlp0_W0, mlp1_W0, mlp2_W0], axis=1).T

    def bd(a, b, c):
        z = jnp.zeros((24, 24), f32)
        return z.at[0:8, 0:8].set(a).at[8:16, 8:16].set(b).at[16:24, 16:24].set(c)

    w1 = bd(mlp0_W1, mlp1_W1, mlp2_W1).T
    w2 = bd(mlp0_W2, mlp1_W2, mlp2_W2).T
    b0 = jnp.concatenate([mlp0_b0, mlp1_b0, mlp2_b0])[:, None]
    b1 = jnp.concatenate([mlp0_b1, mlp1_b1, mlp2_b1])[:, None]
    b2 = jnp.concatenate([mlp0_b2, mlp1_b2, mlp2_b2])[:, None]
    wb = jnp.zeros((MROW, MROW), f32)
    wb = wb.at[0:8, 0:8].set(chan0_W.T)
    wb = wb.at[8:32, 8:32].set(jnp.kron(jnp.eye(3, dtype=f32), chan1_W.T))
    wb = wb.at[32:104, 32:104].set(jnp.kron(jnp.eye(9, dtype=f32), chan2_W.T))

    evp = jnp.concatenate([edge_vector.T,
                           jnp.zeros((5, N_EDGES), f32)], axis=0)

    h0T, pair8 = _sc_gather(i_row, j_row, atom_type, ftab_flat)
    msg = _tc_dense(evp, pair8, h0T, tab, w0, w1, w2, b0, b1, b2)
    i2d = i_row.reshape(NW, NCHUNK, CH)
    zrows = jnp.zeros((N_NODES, MROW), f32)
    part = _sc_scatter(msg, i2d, zrows)
    out = _tc_finish(part, wb)

    M0 = out[:, 0:8].T
    M1 = out[:, 8:32].reshape(N_NODES, 3, 8).transpose(2, 0, 1)
    M2 = out[:, 32:104].reshape(N_NODES, 3, 3, 8).transpose(3, 0, 2, 1)
    return (M0, M1, M2)


# speedup vs baseline: 199.3239x; 1.2220x over previous
import functools

import jax
import jax.numpy as jnp
from jax import lax
from jax.experimental import pallas as pl
from jax.experimental.pallas import tpu as pltpu
from jax.experimental.pallas import tpu_sc as plsc

N_NODES = 10000
N_EDGES = 320000
N_U = 8
N_CHEB = 9
R_CUT = 5.0
NUM_AVG_NEIGH = 32.0

NW = 32
EPT = N_EDGES // NW
CH = 80
NCHUNK = EPT // CH
PCH = 2000
MROW = 128
BLK = 2560
NPTA = 624
NPTL = N_NODES - 15 * NPTA

_SC_PARAMS = pltpu.CompilerParams(needs_layout_passes=False,
                                  use_tc_tiling_on_sc=False)


def _gather_body(i_hbm, j_hbm, type_hbm, ftab_hbm, h0T_hbm, pair8_hbm,
                 type_v, ftab, ibuf, jbuf, sbuf, sem):
    cid = lax.axis_index("c")
    sid = lax.axis_index("s")
    wid = sid * 2 + cid
    base = wid * EPT
    pltpu.sync_copy(type_hbm, type_v)
    pltpu.sync_copy(ftab_hbm, ftab)

    def chunk(c5, _):
        cb = base + c5 * PCH
        pltpu.sync_copy(i_hbm.at[pl.ds(cb, PCH)], ibuf)
        pltpu.sync_copy(j_hbm.at[pl.ds(cb, PCH)], jbuf)

        def grp_pair(g, _):
            iv = ibuf[pl.ds(g * 16, 16)]
            jv = jbuf[pl.ds(g * 16, 16)]
            it = plsc.load_gather(type_v, [iv])
            jt = plsc.load_gather(type_v, [jv])
            pv = (it * 4 + jt).astype(jnp.float32)
            for u in range(8):
                sbuf[u, pl.ds(g * 16, 16)] = pv
            return 0
        lax.fori_loop(0, PCH // 16, grp_pair, 0)
        pltpu.sync_copy(sbuf, pair8_hbm.at[:, pl.ds(cb, PCH)])

        def grp_h0(g, _):
            jv = jbuf[pl.ds(g * 16, 16)]
            j8 = jv * 8
            for u in range(8):
                hv = plsc.load_gather(ftab, [j8 + u])
                sbuf[u, pl.ds(g * 16, 16)] = hv
            return 0
        lax.fori_loop(0, PCH // 16, grp_h0, 0)
        pltpu.sync_copy(sbuf, h0T_hbm.at[:, pl.ds(cb, PCH)])
        return 0
    lax.fori_loop(0, EPT // PCH, chunk, 0)


@jax.jit
def _sc_gather(i_row, j_row, atom_type, ftab_flat):
    mesh = plsc.VectorSubcoreMesh(core_axis_name="c", subcore_axis_name="s")
    fn = pl.kernel(
        _gather_body,
        out_type=(jax.ShapeDtypeStruct((8, N_EDGES), jnp.float32),
                  jax.ShapeDtypeStruct((8, N_EDGES), jnp.float32)),
        mesh=mesh,
        scratch_types=[
            pltpu.VMEM((N_NODES,), jnp.int32),
            pltpu.VMEM((N_NODES * 8,), jnp.float32),
            pltpu.VMEM((PCH,), jnp.int32),
            pltpu.VMEM((PCH,), jnp.int32),
            pltpu.VMEM((8, PCH), jnp.float32),
            pltpu.SemaphoreType.DMA,
        ],
        compiler_params=_SC_PARAMS,
    )
    return fn(i_row, j_row, atom_type, ftab_flat)


def _dense_body(evp, pairb, h0Tb, tab, w0, w1, w2, b0, b1, b2, out):
    x = evp[0:1, :]
    y = evp[1:2, :]
    z = evp[2:3, :]
    r = jnp.sqrt(x * x + y * y + z * z)
    xc = jnp.clip(2.0 * r / R_CUT - 1.0, -1.0, 1.0)
    Ts = [jnp.ones_like(xc), xc]
    for _ in range(2, N_CHEB):
        Ts.append(2.0 * xc * Ts[-1] - Ts[-2])
    Tcm = jnp.concatenate(Ts, axis=0)
    xr = r * (1.0 / R_CUT)
    x2 = xr * xr
    x3 = x2 * xr
    x6 = x3 * x3
    env = jnp.where(xr < 1.0,
                    1.0 - 28.0 * x6 + 48.0 * x6 * xr - 21.0 * x6 * x2,
                    0.0)
    Gm = jnp.dot(tab[...], Tcm, preferred_element_type=jnp.float32)
    pair = pairb[...]
    fu8 = jnp.zeros((8, BLK), jnp.float32)
    for p in range(16):
        fu8 = fu8 + jnp.where(pair == float(p), Gm[8 * p:8 * p + 8, :], 0.0)
    fu = env * fu8
    h1 = jnp.dot(w0[...], fu, preferred_element_type=jnp.float32) + b0[...]
    h1 = h1 * (1.0 / (1.0 + jnp.exp(-h1)))
    h2 = jnp.dot(w1[...], h1, preferred_element_type=jnp.float32) + b1[...]
    h2 = h2 * (1.0 / (1.0 + jnp.exp(-h2)))
    Rm = jnp.dot(w2[...], h2, preferred_element_type=jnp.float32) + b2[...]
    h0T = h0Tb[...]
    Rh = Rm * jnp.concatenate([h0T, h0T, h0T], axis=0)
    rs = jnp.maximum(r, 1e-12)
    unit = evp[0:3, :] / rs
    uu = jnp.concatenate([unit * unit[k:k + 1, :] for k in range(3)], axis=0)
    pieces = [Rh[0:8, :]]
    for a in range(3):
        pieces.append(Rh[8:16, :] * unit[a:a + 1, :])
    for c in range(9):
        pieces.append(Rh[16:24, :] * uu[c:c + 1, :])
    pieces.append(jnp.zeros((MROW - 104, BLK), jnp.float32))
    out[...] = jnp.concatenate(pieces, axis=0).T


def _tc_dense(evp, pair8, h0T, tab, w0, w1, w2, b0, b1, b2):
    nblk = N_EDGES // BLK
    return pl.pallas_call(
        _dense_body,
        grid=(nblk,),
        in_specs=[
            pl.BlockSpec((8, BLK), lambda i: (0, i)),
            pl.BlockSpec((8, BLK), lambda i: (0, i)),
            pl.BlockSpec((8, BLK), lambda i: (0, i)),
            pl.BlockSpec((128, N_CHEB), lambda i: (0, 0)),
            pl.BlockSpec((24, 8), lambda i: (0, 0)),
            pl.BlockSpec((24, 24), lambda i: (0, 0)),
            pl.BlockSpec((24, 24), lambda i: (0, 0)),
            pl.BlockSpec((24, 1), lambda i: (0, 0)),
            pl.BlockSpec((24, 1), lambda i: (0, 0)),
            pl.BlockSpec((24, 1), lambda i: (0, 0)),
        ],
        out_specs=pl.BlockSpec((BLK, MROW), lambda i: (i, 0)),
        out_shape=jax.ShapeDtypeStruct((N_EDGES, MROW), jnp.float32),
    )(evp, pair8, h0T, tab, w0, w1, w2, b0, b1, b2)


def _scatter_body(msg_hbm, i2d_hbm, zrows_hbm, part_hbm, ivm, mb0, mb1, acc,
                  sem0, sem1):
    cid = lax.axis_index("c")
    sid = lax.axis_index("s")
    wid = sid * 2 + cid
    ebase = wid * EPT
    pltpu.sync_copy(i2d_hbm.at[wid], ivm)

    @pl.when(sid < 15)
    def _():
        pltpu.sync_copy(zrows_hbm.at[pl.ds(sid * NPTA, NPTA)],
                        acc.at[pl.ds(sid * NPTA, NPTA)])

    @pl.when(sid == 15)
    def _():
        pltpu.sync_copy(zrows_hbm.at[pl.ds(15 * NPTA, NPTL)],
                        acc.at[pl.ds(15 * NPTA, NPTL)])
    plsc.subcore_barrier()

    def rows(c):
        return pl.ds(ebase + c * CH, CH)

    pltpu.async_copy(msg_hbm.at[rows(0)], mb0, sem0)

    def pairbody(p, _):
        c0 = 2 * p
        c1 = 2 * p + 1
        pltpu.async_copy(msg_hbm.at[rows(c1)], mb1, sem1)
        pltpu.make_async_copy(msg_hbm.at[rows(c0)], mb0, sem0).wait()
        pltpu.sync_copy(mb0, acc.at[ivm.at[c0]], add=True)
        pltpu.async_copy(msg_hbm.at[rows(c0 + 2)], mb0, sem0)
        pltpu.make_async_copy(msg_hbm.at[rows(c1)], mb1, sem1).wait()
        pltpu.sync_copy(mb1, acc.at[ivm.at[c1]], add=True)
        return 0
    lax.fori_loop(0, (NCHUNK - 1) // 2, pairbody, 0)
    pltpu.make_async_copy(msg_hbm.at[rows(NCHUNK - 1)], mb0, sem0).wait()
    pltpu.sync_copy(mb0, acc.at[ivm.at[NCHUNK - 1]], add=True)
    plsc.subcore_barrier()

    @pl.when(sid < 15)
    def _():
        pltpu.sync_copy(acc.at[pl.ds(sid * NPTA, NPTA)],
                        part_hbm.at[cid, pl.ds(sid * NPTA, NPTA)])

    @pl.when(sid == 15)
    def _():
        pltpu.sync_copy(acc.at[pl.ds(15 * NPTA, NPTL)],
                        part_hbm.at[cid, pl.ds(15 * NPTA, NPTL)])


def _sc_scatter(msg, i2d, zrows):
    mesh = plsc.VectorSubcoreMesh(core_axis_name="c", subcore_axis_name="s")
    fn = pl.kernel(
        _scatter_body,
        out_type=jax.ShapeDtypeStruct((2, N_NODES, MROW), jnp.float32),
        mesh=mesh,
        scratch_types=[
            pltpu.VMEM((NCHUNK, CH), jnp.int32),
            pltpu.VMEM((CH, MROW), jnp.float32),
            pltpu.VMEM((CH, MROW), jnp.float32),
            pltpu.VMEM_SHARED((N_NODES, MROW), jnp.float32),
            pltpu.SemaphoreType.DMA,
            pltpu.SemaphoreType.DMA,
        ],
        compiler_params=_SC_PARAMS,
    )
    return fn(msg, i2d, zrows)


def _finish_body(part, wb, out):
    s = (part[0] + part[1]) * (1.0 / (NUM_AVG_NEIGH ** 0.5))
    out[...] = jnp.dot(s, wb[...], preferred_element_type=jnp.float32)


def _tc_finish(part, wb):
    return pl.pallas_call(
        _finish_body,
        out_shape=jax.ShapeDtypeStruct((N_NODES, MROW), jnp.float32),
    )(part, wb)


def kernel(edge_vector, edge_idx, atom_type, atom_feats_0, radial_W,
           mlp0_W0, mlp0_b0, mlp0_W1, mlp0_b1, mlp0_W2, mlp0_b2,
           mlp1_W0, mlp1_b0, mlp1_W1, mlp1_b1, mlp1_W2, mlp1_b2,
           mlp2_W0, mlp2_b0, mlp2_W1, mlp2_b1, mlp2_W2, mlp2_b2,
           chan0_W, chan1_W, chan2_W):
    f32 = jnp.float32
    i_row = edge_idx[0]
    j_row = edge_idx[1]
    ftab_flat = atom_feats_0.T.reshape(N_NODES * 8)

    tab = radial_W.reshape(16, N_U, N_CHEB).reshape(128, N_CHEB)
    w0 = jnp.concatenate([mlp0_W0, mlp1_W0, mlp2_W0], axis=1).T

    def bd(a, b, c):
        z = jnp.zeros((24, 24), f32)
        return z.at[0:8, 0:8].set(a).at[8:16, 8:16].set(b).at[16:24, 16:24].set(c)

    w1 = bd(mlp0_W1, mlp1_W1, mlp2_W1).T
    w2 = bd(mlp0_W2, mlp1_W2, mlp2_W2).T
    b0 = jnp.concatenate([mlp0_b0, mlp1_b0, mlp2_b0])[:, None]
    b1 = jnp.concatenate([mlp0_b1, mlp1_b1, mlp2_b1])[:, None]
    b2 = jnp.concatenate([mlp0_b2, mlp1_b2, mlp2_b2])[:, None]
    wb = jnp.zeros((MROW, MROW), f32)
    wb = wb.at[0:8, 0:8].set(chan0_W.T)
    wb = wb.at[8:32, 8:32].set(jnp.kron(jnp.eye(3, dtype=f32), chan1_W.T))
    wb = wb.at[32:104, 32:104].set(jnp.kron(jnp.eye(9, dtype=f32), chan2_W.T))

    evp = jnp.concatenate([edge_vector.T,
                           jnp.zeros((5, N_EDGES), f32)], axis=0)

    h0T, pair8 = _sc_gather(i_row, j_row, atom_type, ftab_flat)
    msg = _tc_dense(evp, pair8, h0T, tab, w0, w1, w2, b0, b1, b2)
    i2d = i_row.reshape(NW, NCHUNK, CH)
    zrows = jnp.zeros((N_NODES, MROW), f32)
    part = _sc_scatter(msg, i2d, zrows)
    out = _tc_finish(part, wb)

    M0 = out[:, 0:8].T
    M1 = out[:, 8:32].reshape(N_NODES, 3, 8).transpose(2, 0, 1)
    M2 = out[:, 32:104].reshape(N_NODES, 3, 3, 8).transpose(3, 0, 2, 1)
    return (M0, M1, M2)


# speedup vs baseline: 222.9007x; 1.1183x over previous
import functools

import jax
import jax.numpy as jnp
from jax import lax
from jax.experimental import pallas as pl
from jax.experimental.pallas import tpu as pltpu
from jax.experimental.pallas import tpu_sc as plsc

N_NODES = 10000
N_EDGES = 320000
N_U = 8
N_CHEB = 9
R_CUT = 5.0
NUM_AVG_NEIGH = 32.0

NW = 32
EPT = N_EDGES // NW
CH = 80
NCHUNK = EPT // CH
PCH = 2000
MROW = 128
BLK = 6400
NPTA = 624
NPTL = N_NODES - 15 * NPTA

_SC_PARAMS = pltpu.CompilerParams(needs_layout_passes=False,
                                  use_tc_tiling_on_sc=False)


def _gather_body(i_hbm, j_hbm, type_hbm, ftab_hbm, h0T_hbm, pair8_hbm,
                 type_v, ftab, ibuf, jbuf, sbuf, sem):
    cid = lax.axis_index("c")
    sid = lax.axis_index("s")
    wid = sid * 2 + cid
    base = wid * EPT
    pltpu.sync_copy(type_hbm, type_v)
    pltpu.sync_copy(ftab_hbm, ftab)

    def chunk(c5, _):
        cb = base + c5 * PCH
        pltpu.sync_copy(i_hbm.at[pl.ds(cb, PCH)], ibuf)
        pltpu.sync_copy(j_hbm.at[pl.ds(cb, PCH)], jbuf)

        def grp_pair(g, _):
            iv = ibuf[pl.ds(g * 16, 16)]
            jv = jbuf[pl.ds(g * 16, 16)]
            it = plsc.load_gather(type_v, [iv])
            jt = plsc.load_gather(type_v, [jv])
            pv = (it * 4 + jt).astype(jnp.float32)
            for u in range(8):
                sbuf[u, pl.ds(g * 16, 16)] = pv
            return 0
        lax.fori_loop(0, PCH // 16, grp_pair, 0)
        pltpu.sync_copy(sbuf, pair8_hbm.at[:, pl.ds(cb, PCH)])

        def grp_h0(g, _):
            jv = jbuf[pl.ds(g * 16, 16)]
            j8 = jv * 8
            for u in range(8):
                hv = plsc.load_gather(ftab, [j8 + u])
                sbuf[u, pl.ds(g * 16, 16)] = hv
            return 0
        lax.fori_loop(0, PCH // 16, grp_h0, 0)
        pltpu.sync_copy(sbuf, h0T_hbm.at[:, pl.ds(cb, PCH)])
        return 0
    lax.fori_loop(0, EPT // PCH, chunk, 0)


@jax.jit
def _sc_gather(i_row, j_row, atom_type, ftab_flat):
    mesh = plsc.VectorSubcoreMesh(core_axis_name="c", subcore_axis_name="s")
    fn = pl.kernel(
        _gather_body,
        out_type=(jax.ShapeDtypeStruct((8, N_EDGES), jnp.float32),
                  jax.ShapeDtypeStruct((8, N_EDGES), jnp.float32)),
        mesh=mesh,
        scratch_types=[
            pltpu.VMEM((N_NODES,), jnp.int32),
            pltpu.VMEM((N_NODES * 8,), jnp.float32),
            pltpu.VMEM((PCH,), jnp.int32),
            pltpu.VMEM((PCH,), jnp.int32),
            pltpu.VMEM((8, PCH), jnp.float32),
            pltpu.SemaphoreType.DMA,
        ],
        compiler_params=_SC_PARAMS,
    )
    return fn(i_row, j_row, atom_type, ftab_flat)


def _dense_body(evp, pairb, h0Tb, tab, w0, w1, w2, b0, b1, b2, out):
    x = evp[0:1, :]
    y = evp[1:2, :]
    z = evp[2:3, :]
    r = jnp.sqrt(x * x + y * y + z * z)
    xc = jnp.clip(2.0 * r / R_CUT - 1.0, -1.0, 1.0)
    Ts = [jnp.ones_like(xc), xc]
    for _ in range(2, N_CHEB):
        Ts.append(2.0 * xc * Ts[-1] - Ts[-2])
    Tcm = jnp.concatenate(Ts, axis=0)
    xr = r * (1.0 / R_CUT)
    x2 = xr * xr
    x3 = x2 * xr
    x6 = x3 * x3
    env = jnp.where(xr < 1.0,
                    1.0 - 28.0 * x6 + 48.0 * x6 * xr - 21.0 * x6 * x2,
                    0.0)
    Gm = jnp.dot(tab[...], Tcm, preferred_element_type=jnp.float32)
    pair = pairb[...]
    fu8 = jnp.zeros((8, BLK), jnp.float32)
    for p in range(16):
        fu8 = fu8 + jnp.where(pair == float(p), Gm[8 * p:8 * p + 8, :], 0.0)
    fu = env * fu8
    h1 = jnp.dot(w0[...], fu, preferred_element_type=jnp.float32) + b0[...]
    h1 = h1 * (1.0 / (1.0 + jnp.exp(-h1)))
    h2 = jnp.dot(w1[...], h1, preferred_element_type=jnp.float32) + b1[...]
    h2 = h2 * (1.0 / (1.0 + jnp.exp(-h2)))
    Rm = jnp.dot(w2[...], h2, preferred_element_type=jnp.float32) + b2[...]
    h0T = h0Tb[...]
    Rh = Rm * jnp.concatenate([h0T, h0T, h0T], axis=0)
    rs = jnp.maximum(r, 1e-12)
    unit = evp[0:3, :] / rs
    uu = jnp.concatenate([unit * unit[k:k + 1, :] for k in range(3)], axis=0)
    pieces = [Rh[0:8, :]]
    for a in range(3):
        pieces.append(Rh[8:16, :] * unit[a:a + 1, :])
    for c in range(9):
        pieces.append(Rh[16:24, :] * uu[c:c + 1, :])
    pieces.append(jnp.zeros((MROW - 104, BLK), jnp.float32))
    out[...] = jnp.concatenate(pieces, axis=0).T


def _tc_dense(evp, pair8, h0T, tab, w0, w1, w2, b0, b1, b2):
    nblk = N_EDGES // BLK
    return pl.pallas_call(
        _dense_body,
        grid=(nblk,),
        in_specs=[
            pl.BlockSpec((8, BLK), lambda i: (0, i)),
            pl.BlockSpec((8, BLK), lambda i: (0, i)),
            pl.BlockSpec((8, BLK), lambda i: (0, i)),
            pl.BlockSpec((128, N_CHEB), lambda i: (0, 0)),
            pl.BlockSpec((24, 8), lambda i: (0, 0)),
            pl.BlockSpec((24, 24), lambda i: (0, 0)),
            pl.BlockSpec((24, 24), lambda i: (0, 0)),
            pl.BlockSpec((24, 1), lambda i: (0, 0)),
            pl.BlockSpec((24, 1), lambda i: (0, 0)),
            pl.BlockSpec((24, 1), lambda i: (0, 0)),
        ],
        out_specs=pl.BlockSpec((BLK, MROW), lambda i: (i, 0)),
        out_shape=jax.ShapeDtypeStruct((N_EDGES, MROW), jnp.float32),
    )(evp, pair8, h0T, tab, w0, w1, w2, b0, b1, b2)


def _scatter_body(msg_hbm, i2d_hbm, zrows_hbm, part_hbm, ivm, mb0, mb1, acc,
                  sem0, sem1):
    cid = lax.axis_index("c")
    sid = lax.axis_index("s")
    wid = sid * 2 + cid
    ebase = wid * EPT
    pltpu.sync_copy(i2d_hbm.at[wid], ivm)

    @pl.when(sid < 15)
    def _():
        pltpu.sync_copy(zrows_hbm.at[pl.ds(sid * NPTA, NPTA)],
                        acc.at[pl.ds(sid * NPTA, NPTA)])

    @pl.when(sid == 15)
    def _():
        pltpu.sync_copy(zrows_hbm.at[pl.ds(15 * NPTA, NPTL)],
                        acc.at[pl.ds(15 * NPTA, NPTL)])
    plsc.subcore_barrier()

    def rows(c):
        return pl.ds(ebase + c * CH, CH)

    pltpu.async_copy(msg_hbm.at[rows(0)], mb0, sem0)

    def pairbody(p, _):
        c0 = 2 * p
        c1 = 2 * p + 1
        pltpu.async_copy(msg_hbm.at[rows(c1)], mb1, sem1)
        pltpu.make_async_copy(msg_hbm.at[rows(c0)], mb0, sem0).wait()
        pltpu.sync_copy(mb0, acc.at[ivm.at[c0]], add=True)
        pltpu.async_copy(msg_hbm.at[rows(c0 + 2)], mb0, sem0)
        pltpu.make_async_copy(msg_hbm.at[rows(c1)], mb1, sem1).wait()
        pltpu.sync_copy(mb1, acc.at[ivm.at[c1]], add=True)
        return 0
    lax.fori_loop(0, (NCHUNK - 1) // 2, pairbody, 0)
    pltpu.make_async_copy(msg_hbm.at[rows(NCHUNK - 1)], mb0, sem0).wait()
    pltpu.sync_copy(mb0, acc.at[ivm.at[NCHUNK - 1]], add=True)
    plsc.subcore_barrier()

    @pl.when(sid < 15)
    def _():
        pltpu.sync_copy(acc.at[pl.ds(sid * NPTA, NPTA)],
                        part_hbm.at[cid, pl.ds(sid * NPTA, NPTA)])

    @pl.when(sid == 15)
    def _():
        pltpu.sync_copy(acc.at[pl.ds(15 * NPTA, NPTL)],
                        part_hbm.at[cid, pl.ds(15 * NPTA, NPTL)])


def _sc_scatter(msg, i2d, zrows):
    mesh = plsc.VectorSubcoreMesh(core_axis_name="c", subcore_axis_name="s")
    fn = pl.kernel(
        _scatter_body,
        out_type=jax.ShapeDtypeStruct((2, N_NODES, MROW), jnp.float32),
        mesh=mesh,
        scratch_types=[
            pltpu.VMEM((NCHUNK, CH), jnp.int32),
            pltpu.VMEM((CH, MROW), jnp.float32),
            pltpu.VMEM((CH, MROW), jnp.float32),
            pltpu.VMEM_SHARED((N_NODES, MROW), jnp.float32),
            pltpu.SemaphoreType.DMA,
            pltpu.SemaphoreType.DMA,
        ],
        compiler_params=_SC_PARAMS,
    )
    return fn(msg, i2d, zrows)


def _finish_body(part, wb, out):
    s = (part[0] + part[1]) * (1.0 / (NUM_AVG_NEIGH ** 0.5))
    out[...] = jnp.dot(s, wb[...], preferred_element_type=jnp.float32)


def _tc_finish(part, wb):
    return pl.pallas_call(
        _finish_body,
        out_shape=jax.ShapeDtypeStruct((N_NODES, MROW), jnp.float32),
    )(part, wb)


def kernel(edge_vector, edge_idx, atom_type, atom_feats_0, radial_W,
           mlp0_W0, mlp0_b0, mlp0_W1, mlp0_b1, mlp0_W2, mlp0_b2,
           mlp1_W0, mlp1_b0, mlp1_W1, mlp1_b1, mlp1_W2, mlp1_b2,
           mlp2_W0, mlp2_b0, mlp2_W1, mlp2_b1, mlp2_W2, mlp2_b2,
           chan0_W, chan1_W, chan2_W):
    f32 = jnp.float32
    i_row = edge_idx[0]
    j_row = edge_idx[1]
    ftab_flat = atom_feats_0.T.reshape(N_NODES * 8)

    tab = radial_W.reshape(16, N_U, N_CHEB).reshape(128, N_CHEB)
    w0 = jnp.concatenate([mlp0_W0, mlp1_W0, mlp2_W0], axis=1).T

    def bd(a, b, c):
        z = jnp.zeros((24, 24), f32)
        return z.at[0:8, 0:8].set(a).at[8:16, 8:16].set(b).at[16:24, 16:24].set(c)

    w1 = bd(mlp0_W1, mlp1_W1, mlp2_W1).T
    w2 = bd(mlp0_W2, mlp1_W2, mlp2_W2).T
    b0 = jnp.concatenate([mlp0_b0, mlp1_b0, mlp2_b0])[:, None]
    b1 = jnp.concatenate([mlp0_b1, mlp1_b1, mlp2_b1])[:, None]
    b2 = jnp.concatenate([mlp0_b2, mlp1_b2, mlp2_b2])[:, None]
    wb = jnp.zeros((MROW, MROW), f32)
    wb = wb.at[0:8, 0:8].set(chan0_W.T)
    wb = wb.at[8:32, 8:32].set(jnp.kron(jnp.eye(3, dtype=f32), chan1_W.T))
    wb = wb.at[32:104, 32:104].set(jnp.kron(jnp.eye(9, dtype=f32), chan2_W.T))

    evp = jnp.concatenate([edge_vector.T,
                           jnp.zeros((5, N_EDGES), f32)], axis=0)

    h0T, pair8 = _sc_gather(i_row, j_row, atom_type, ftab_flat)
    msg = _tc_dense(evp, pair8, h0T, tab, w0, w1, w2, b0, b1, b2)
    i2d = i_row.reshape(NW, NCHUNK, CH)
    zrows = jnp.zeros((N_NODES, MROW), f32)
    part = _sc_scatter(msg, i2d, zrows)
    out = _tc_finish(part, wb)

    M0 = out[:, 0:8].T
    M1 = out[:, 8:32].reshape(N_NODES, 3, 8).transpose(2, 0, 1)
    M2 = out[:, 32:104].reshape(N_NODES, 3, 3, 8).transpose(3, 0, 2, 1)
    return (M0, M1, M2)


# speedup vs baseline: 224.9412x; 1.0092x over previous
import functools

import jax
import jax.numpy as jnp
from jax import lax
from jax.experimental import pallas as pl
from jax.experimental.pallas import tpu as pltpu
from jax.experimental.pallas import tpu_sc as plsc

N_NODES = 10000
N_EDGES = 320000
N_U = 8
N_CHEB = 9
R_CUT = 5.0
NUM_AVG_NEIGH = 32.0

NW = 32
EPT = N_EDGES // NW
CH = 80
NCHUNK = EPT // CH
PCH = 2000
MROW = 128
BLK = 12800
NPTA = 624
NPTL = N_NODES - 15 * NPTA

_SC_PARAMS = pltpu.CompilerParams(needs_layout_passes=False,
                                  use_tc_tiling_on_sc=False)


def _gather_body(i_hbm, j_hbm, type_hbm, ftab_hbm, h0T_hbm, pair8_hbm,
                 type_v, ftab, ibuf, jbuf, sbuf, sem):
    cid = lax.axis_index("c")
    sid = lax.axis_index("s")
    wid = sid * 2 + cid
    base = wid * EPT
    pltpu.sync_copy(type_hbm, type_v)
    pltpu.sync_copy(ftab_hbm, ftab)

    def chunk(c5, _):
        cb = base + c5 * PCH
        pltpu.sync_copy(i_hbm.at[pl.ds(cb, PCH)], ibuf)
        pltpu.sync_copy(j_hbm.at[pl.ds(cb, PCH)], jbuf)

        def grp_pair(g, _):
            iv = ibuf[pl.ds(g * 16, 16)]
            jv = jbuf[pl.ds(g * 16, 16)]
            it = plsc.load_gather(type_v, [iv])
            jt = plsc.load_gather(type_v, [jv])
            pv = (it * 4 + jt).astype(jnp.float32)
            for u in range(8):
                sbuf[u, pl.ds(g * 16, 16)] = pv
            return 0
        lax.fori_loop(0, PCH // 16, grp_pair, 0)
        pltpu.sync_copy(sbuf, pair8_hbm.at[:, pl.ds(cb, PCH)])

        def grp_h0(g, _):
            jv = jbuf[pl.ds(g * 16, 16)]
            j8 = jv * 8
            for u in range(8):
                hv = plsc.load_gather(ftab, [j8 + u])
                sbuf[u, pl.ds(g * 16, 16)] = hv
            return 0
        lax.fori_loop(0, PCH // 16, grp_h0, 0)
        pltpu.sync_copy(sbuf, h0T_hbm.at[:, pl.ds(cb, PCH)])
        return 0
    lax.fori_loop(0, EPT // PCH, chunk, 0)


@jax.jit
def _sc_gather(i_row, j_row, atom_type, ftab_flat):
    mesh = plsc.VectorSubcoreMesh(core_axis_name="c", subcore_axis_name="s")
    fn = pl.kernel(
        _gather_body,
        out_type=(jax.ShapeDtypeStruct((8, N_EDGES), jnp.float32),
                  jax.ShapeDtypeStruct((8, N_EDGES), jnp.float32)),
        mesh=mesh,
        scratch_types=[
            pltpu.VMEM((N_NODES,), jnp.int32),
            pltpu.VMEM((N_NODES * 8,), jnp.float32),
            pltpu.VMEM((PCH,), jnp.int32),
            pltpu.VMEM((PCH,), jnp.int32),
            pltpu.VMEM((8, PCH), jnp.float32),
            pltpu.SemaphoreType.DMA,
        ],
        compiler_params=_SC_PARAMS,
    )
    return fn(i_row, j_row, atom_type, ftab_flat)


def _dense_body(evp, pairb, h0Tb, tab, w0, w1, w2, b0, b1, b2, out):
    x = evp[0:1, :]
    y = evp[1:2, :]
    z = evp[2:3, :]
    r = jnp.sqrt(x * x + y * y + z * z)
    xc = jnp.clip(2.0 * r / R_CUT - 1.0, -1.0, 1.0)
    Ts = [jnp.ones_like(xc), xc]
    for _ in range(2, N_CHEB):
        Ts.append(2.0 * xc * Ts[-1] - Ts[-2])
    Tcm = jnp.concatenate(Ts, axis=0)
    xr = r * (1.0 / R_CUT)
    x2 = xr * xr
    x3 = x2 * xr
    x6 = x3 * x3
    env = jnp.where(xr < 1.0,
                    1.0 - 28.0 * x6 + 48.0 * x6 * xr - 21.0 * x6 * x2,
                    0.0)
    Gm = jnp.dot(tab[...], Tcm, preferred_element_type=jnp.float32)
    pair = pairb[...]
    fu8 = jnp.zeros((8, BLK), jnp.float32)
    for p in range(16):
        fu8 = fu8 + jnp.where(pair == float(p), Gm[8 * p:8 * p + 8, :], 0.0)
    fu = env * fu8
    h1 = jnp.dot(w0[...], fu, preferred_element_type=jnp.float32) + b0[...]
    h1 = h1 * (1.0 / (1.0 + jnp.exp(-h1)))
    h2 = jnp.dot(w1[...], h1, preferred_element_type=jnp.float32) + b1[...]
    h2 = h2 * (1.0 / (1.0 + jnp.exp(-h2)))
    Rm = jnp.dot(w2[...], h2, preferred_element_type=jnp.float32) + b2[...]
    h0T = h0Tb[...]
    Rh = Rm * jnp.concatenate([h0T, h0T, h0T], axis=0)
    rs = jnp.maximum(r, 1e-12)
    unit = evp[0:3, :] / rs
    uu = jnp.concatenate([unit * unit[k:k + 1, :] for k in range(3)], axis=0)
    pieces = [Rh[0:8, :]]
    for a in range(3):
        pieces.append(Rh[8:16, :] * unit[a:a + 1, :])
    for c in range(9):
        pieces.append(Rh[16:24, :] * uu[c:c + 1, :])
    pieces.append(jnp.zeros((MROW - 104, BLK), jnp.float32))
    out[...] = jnp.concatenate(pieces, axis=0).T


def _tc_dense(evp, pair8, h0T, tab, w0, w1, w2, b0, b1, b2):
    nblk = N_EDGES // BLK
    return pl.pallas_call(
        _dense_body,
        grid=(nblk,),
        in_specs=[
            pl.BlockSpec((8, BLK), lambda i: (0, i)),
            pl.BlockSpec((8, BLK), lambda i: (0, i)),
            pl.BlockSpec((8, BLK), lambda i: (0, i)),
            pl.BlockSpec((128, N_CHEB), lambda i: (0, 0)),
            pl.BlockSpec((24, 8), lambda i: (0, 0)),
            pl.BlockSpec((24, 24), lambda i: (0, 0)),
            pl.BlockSpec((24, 24), lambda i: (0, 0)),
            pl.BlockSpec((24, 1), lambda i: (0, 0)),
            pl.BlockSpec((24, 1), lambda i: (0, 0)),
            pl.BlockSpec((24, 1), lambda i: (0, 0)),
        ],
        out_specs=pl.BlockSpec((BLK, MROW), lambda i: (i, 0)),
        out_shape=jax.ShapeDtypeStruct((N_EDGES, MROW), jnp.float32),
    )(evp, pair8, h0T, tab, w0, w1, w2, b0, b1, b2)


def _scatter_body(msg_hbm, i2d_hbm, zrows_hbm, part_hbm, ivm, mb0, mb1, acc,
                  sem0, sem1):
    cid = lax.axis_index("c")
    sid = lax.axis_index("s")
    wid = sid * 2 + cid
    ebase = wid * EPT
    pltpu.sync_copy(i2d_hbm.at[wid], ivm)

    @pl.when(sid < 15)
    def _():
        pltpu.sync_copy(zrows_hbm.at[pl.ds(sid * NPTA, NPTA)],
                        acc.at[pl.ds(sid * NPTA, NPTA)])

    @pl.when(sid == 15)
    def _():
        pltpu.sync_copy(zrows_hbm.at[pl.ds(15 * NPTA, NPTL)],
                        acc.at[pl.ds(15 * NPTA, NPTL)])
    plsc.subcore_barrier()

    def rows(c):
        return pl.ds(ebase + c * CH, CH)

    pltpu.async_copy(msg_hbm.at[rows(0)], mb0, sem0)

    def pairbody(p, _):
        c0 = 2 * p
        c1 = 2 * p + 1
        pltpu.async_copy(msg_hbm.at[rows(c1)], mb1, sem1)
        pltpu.make_async_copy(msg_hbm.at[rows(c0)], mb0, sem0).wait()
        pltpu.sync_copy(mb0, acc.at[ivm.at[c0]], add=True)
        pltpu.async_copy(msg_hbm.at[rows(c0 + 2)], mb0, sem0)
        pltpu.make_async_copy(msg_hbm.at[rows(c1)], mb1, sem1).wait()
        pltpu.sync_copy(mb1, acc.at[ivm.at[c1]], add=True)
        return 0
    lax.fori_loop(0, (NCHUNK - 1) // 2, pairbody, 0)
    pltpu.make_async_copy(msg_hbm.at[rows(NCHUNK - 1)], mb0, sem0).wait()
    pltpu.sync_copy(mb0, acc.at[ivm.at[NCHUNK - 1]], add=True)
    plsc.subcore_barrier()

    @pl.when(sid < 15)
    def _():
        pltpu.sync_copy(acc.at[pl.ds(sid * NPTA, NPTA)],
                        part_hbm.at[cid, pl.ds(sid * NPTA, NPTA)])

    @pl.when(sid == 15)
    def _():
        pltpu.sync_copy(acc.at[pl.ds(15 * NPTA, NPTL)],
                        part_hbm.at[cid, pl.ds(15 * NPTA, NPTL)])


def _sc_scatter(msg, i2d, zrows):
    mesh = plsc.VectorSubcoreMesh(core_axis_name="c", subcore_axis_name="s")
    fn = pl.kernel(
        _scatter_body,
        out_type=jax.ShapeDtypeStruct((2, N_NODES, MROW), jnp.float32),
        mesh=mesh,
        scratch_types=[
            pltpu.VMEM((NCHUNK, CH), jnp.int32),
            pltpu.VMEM((CH, MROW), jnp.float32),
            pltpu.VMEM((CH, MROW), jnp.float32),
            pltpu.VMEM_SHARED((N_NODES, MROW), jnp.float32),
            pltpu.SemaphoreType.DMA,
            pltpu.SemaphoreType.DMA,
        ],
        compiler_params=_SC_PARAMS,
    )
    return fn(msg, i2d, zrows)


def _finish_body(part, wb, out):
    s = (part[0] + part[1]) * (1.0 / (NUM_AVG_NEIGH ** 0.5))
    out[...] = jnp.dot(s, wb[...], preferred_element_type=jnp.float32)


def _tc_finish(part, wb):
    return pl.pallas_call(
        _finish_body,
        out_shape=jax.ShapeDtypeStruct((N_NODES, MROW), jnp.float32),
    )(part, wb)


def kernel(edge_vector, edge_idx, atom_type, atom_feats_0, radial_W,
           mlp0_W0, mlp0_b0, mlp0_W1, mlp0_b1, mlp0_W2, mlp0_b2,
           mlp1_W0, mlp1_b0, mlp1_W1, mlp1_b1, mlp1_W2, mlp1_b2,
           mlp2_W0, mlp2_b0, mlp2_W1, mlp2_b1, mlp2_W2, mlp2_b2,
           chan0_W, chan1_W, chan2_W):
    f32 = jnp.float32
    i_row = edge_idx[0]
    j_row = edge_idx[1]
    ftab_flat = atom_feats_0.T.reshape(N_NODES * 8)

    tab = radial_W.reshape(16, N_U, N_CHEB).reshape(128, N_CHEB)
    w0 = jnp.concatenate([mlp0_W0, mlp1_W0, mlp2_W0], axis=1).T

    def bd(a, b, c):
        z = jnp.zeros((24, 24), f32)
        return z.at[0:8, 0:8].set(a).at[8:16, 8:16].set(b).at[16:24, 16:24].set(c)

    w1 = bd(mlp0_W1, mlp1_W1, mlp2_W1).T
    w2 = bd(mlp0_W2, mlp1_W2, mlp2_W2).T
    b0 = jnp.concatenate([mlp0_b0, mlp1_b0, mlp2_b0])[:, None]
    b1 = jnp.concatenate([mlp0_b1, mlp1_b1, mlp2_b1])[:, None]
    b2 = jnp.concatenate([mlp0_b2, mlp1_b2, mlp2_b2])[:, None]
    wb = jnp.zeros((MROW, MROW), f32)
    wb = wb.at[0:8, 0:8].set(chan0_W.T)
    wb = wb.at[8:32, 8:32].set(jnp.kron(jnp.eye(3, dtype=f32), chan1_W.T))
    wb = wb.at[32:104, 32:104].set(jnp.kron(jnp.eye(9, dtype=f32), chan2_W.T))

    evp = jnp.concatenate([edge_vector.T,
                           jnp.zeros((5, N_EDGES), f32)], axis=0)

    h0T, pair8 = _sc_gather(i_row, j_row, atom_type, ftab_flat)
    msg = _tc_dense(evp, pair8, h0T, tab, w0, w1, w2, b0, b1, b2)
    i2d = i_row.reshape(NW, NCHUNK, CH)
    zrows = jnp.zeros((N_NODES, MROW), f32)
    part = _sc_scatter(msg, i2d, zrows)
    out = _tc_finish(part, wb)

    M0 = out[:, 0:8].T
    M1 = out[:, 8:32].reshape(N_NODES, 3, 8).transpose(2, 0, 1)
    M2 = out[:, 32:104].reshape(N_NODES, 3, 3, 8).transpose(3, 0, 2, 1)
    return (M0, M1, M2)


# speedup vs baseline: 245.2012x; 1.0901x over previous
import functools

import jax
import jax.numpy as jnp
from jax import lax
from jax.experimental import pallas as pl
from jax.experimental.pallas import tpu as pltpu
from jax.experimental.pallas import tpu_sc as plsc

N_NODES = 10000
N_EDGES = 320000
N_U = 8
N_CHEB = 9
R_CUT = 5.0
NUM_AVG_NEIGH = 32.0

NW = 32
CH = 80
MROW = 128
BLK = 6400
NPTA = 624
NPTL = N_NODES - 15 * NPTA

PT_A = 4800
PT_B = 5200
HALF_A = NW * PT_A
HALF_B = NW * PT_B
CHUNKS_A = ((0, 1600), (1600, 1600), (3200, 1600))
CHUNKS_B = ((0, 1600), (1600, 1600), (3200, 1600), (4800, 400))
PCHMAX = 1600

_SC_PARAMS = pltpu.CompilerParams(needs_layout_passes=False,
                                  use_tc_tiling_on_sc=False)


def _gather_body(pt, chunks, i_hbm, j_hbm, type_hbm, ftab_hbm, h0T_hbm,
                 pair8_hbm, type_v, ftab, ibuf, jbuf, sbuf, sem):
    cid = lax.axis_index("c")
    sid = lax.axis_index("s")
    wid = sid * 2 + cid
    base = wid * pt
    pltpu.sync_copy(type_hbm, type_v)
    pltpu.sync_copy(ftab_hbm, ftab)

    for (off, sz) in chunks:
        cb = base + off
        pltpu.sync_copy(i_hbm.at[pl.ds(cb, sz)], ibuf.at[pl.ds(0, sz)])
        pltpu.sync_copy(j_hbm.at[pl.ds(cb, sz)], jbuf.at[pl.ds(0, sz)])

        def grp_pair(g, _):
            iv = ibuf[pl.ds(g * 16, 16)]
            jv = jbuf[pl.ds(g * 16, 16)]
            it = plsc.load_gather(type_v, [iv])
            jt = plsc.load_gather(type_v, [jv])
            pv = (it * 4 + jt).astype(jnp.float32)
            for u in range(8):
                sbuf[u, pl.ds(g * 16, 16)] = pv
            return 0
        lax.fori_loop(0, sz // 16, grp_pair, 0)
        pltpu.sync_copy(sbuf.at[:, pl.ds(0, sz)],
                        pair8_hbm.at[:, pl.ds(cb, sz)])

        def grp_h0(g, _):
            jv = jbuf[pl.ds(g * 16, 16)]
            j8 = jv * 8
            for u in range(8):
                hv = plsc.load_gather(ftab, [j8 + u])
                sbuf[u, pl.ds(g * 16, 16)] = hv
            return 0
        lax.fori_loop(0, sz // 16, grp_h0, 0)
        pltpu.sync_copy(sbuf.at[:, pl.ds(0, sz)],
                        h0T_hbm.at[:, pl.ds(cb, sz)])


def _sc_gather(i_row, j_row, atom_type, ftab_flat, pt, chunks, half):
    mesh = plsc.VectorSubcoreMesh(core_axis_name="c", subcore_axis_name="s")
    fn = pl.kernel(
        functools.partial(_gather_body, pt, chunks),
        out_type=(jax.ShapeDtypeStruct((8, half), jnp.float32),
                  jax.ShapeDtypeStruct((8, half), jnp.float32)),
        mesh=mesh,
        scratch_types=[
            pltpu.VMEM((N_NODES,), jnp.int32),
            pltpu.VMEM((N_NODES * 8,), jnp.float32),
            pltpu.VMEM((PCHMAX,), jnp.int32),
            pltpu.VMEM((PCHMAX,), jnp.int32),
            pltpu.VMEM((8, PCHMAX), jnp.float32),
            pltpu.SemaphoreType.DMA,
        ],
        compiler_params=_SC_PARAMS,
    )
    return fn(i_row, j_row, atom_type, ftab_flat)


def _dense_body(evp, pairb, h0Tb, tab, w0, w1, w2, b0, b1, b2, out):
    x = evp[0:1, :]
    y = evp[1:2, :]
    z = evp[2:3, :]
    r = jnp.sqrt(x * x + y * y + z * z)
    xc = jnp.clip(2.0 * r / R_CUT - 1.0, -1.0, 1.0)
    Ts = [jnp.ones_like(xc), xc]
    for _ in range(2, N_CHEB):
        Ts.append(2.0 * xc * Ts[-1] - Ts[-2])
    Tcm = jnp.concatenate(Ts, axis=0)
    xr = r * (1.0 / R_CUT)
    x2 = xr * xr
    x3 = x2 * xr
    x6 = x3 * x3
    env = jnp.where(xr < 1.0,
                    1.0 - 28.0 * x6 + 48.0 * x6 * xr - 21.0 * x6 * x2,
                    0.0)
    Gm = jnp.dot(tab[...], Tcm, preferred_element_type=jnp.float32)
    pair = pairb[...]
    fu8 = jnp.zeros((8, BLK), jnp.float32)
    for p in range(16):
        fu8 = fu8 + jnp.where(pair == float(p), Gm[8 * p:8 * p + 8, :], 0.0)
    fu = env * fu8
    h1 = jnp.dot(w0[...], fu, preferred_element_type=jnp.float32) + b0[...]
    h1 = h1 * (1.0 / (1.0 + jnp.exp(-h1)))
    h2 = jnp.dot(w1[...], h1, preferred_element_type=jnp.float32) + b1[...]
    h2 = h2 * (1.0 / (1.0 + jnp.exp(-h2)))
    Rm = jnp.dot(w2[...], h2, preferred_element_type=jnp.float32) + b2[...]
    h0T = h0Tb[...]
    Rh = Rm * jnp.concatenate([h0T, h0T, h0T], axis=0)
    rs = jnp.maximum(r, 1e-12)
    unit = evp[0:3, :] / rs
    uu = jnp.concatenate([unit * unit[k:k + 1, :] for k in range(3)], axis=0)
    pieces = [Rh[0:8, :]]
    for a in range(3):
        pieces.append(Rh[8:16, :] * unit[a:a + 1, :])
    for c in range(9):
        pieces.append(Rh[16:24, :] * uu[c:c + 1, :])
    pieces.append(jnp.zeros((MROW - 104, BLK), jnp.float32))
    out[...] = jnp.concatenate(pieces, axis=0).T


def _tc_dense(evp, pair8, h0T, tab, w0, w1, w2, b0, b1, b2, half):
    nblk = half // BLK
    return pl.pallas_call(
        _dense_body,
        grid=(nblk,),
        in_specs=[
            pl.BlockSpec((8, BLK), lambda i: (0, i)),
            pl.BlockSpec((8, BLK), lambda i: (0, i)),
            pl.BlockSpec((8, BLK), lambda i: (0, i)),
            pl.BlockSpec((128, N_CHEB), lambda i: (0, 0)),
            pl.BlockSpec((24, 8), lambda i: (0, 0)),
            pl.BlockSpec((24, 24), lambda i: (0, 0)),
            pl.BlockSpec((24, 24), lambda i: (0, 0)),
            pl.BlockSpec((24, 1), lambda i: (0, 0)),
            pl.BlockSpec((24, 1), lambda i: (0, 0)),
            pl.BlockSpec((24, 1), lambda i: (0, 0)),
        ],
        out_specs=pl.BlockSpec((BLK, MROW), lambda i: (i, 0)),
        out_shape=jax.ShapeDtypeStruct((half, MROW), jnp.float32),
    )(evp, pair8, h0T, tab, w0, w1, w2, b0, b1, b2)


def _scatter_body(pt, nch, msg_hbm, i2d_hbm, zrows_hbm, part_hbm, ivm, mb0,
                  mb1, acc, sem0, sem1):
    cid = lax.axis_index("c")
    sid = lax.axis_index("s")
    wid = sid * 2 + cid
    ebase = wid * pt
    pltpu.sync_copy(i2d_hbm.at[wid], ivm)

    @pl.when(sid < 15)
    def _():
        pltpu.sync_copy(zrows_hbm.at[pl.ds(sid * NPTA, NPTA)],
                        acc.at[pl.ds(sid * NPTA, NPTA)])

    @pl.when(sid == 15)
    def _():
        pltpu.sync_copy(zrows_hbm.at[pl.ds(15 * NPTA, NPTL)],
                        acc.at[pl.ds(15 * NPTA, NPTL)])
    plsc.subcore_barrier()

    def rows(c):
        return pl.ds(ebase + c * CH, CH)

    pltpu.async_copy(msg_hbm.at[rows(0)], mb0, sem0)
    pltpu.async_copy(msg_hbm.at[rows(1)], mb1, sem1)

    def pairbody(p, _):
        c0 = 2 * p
        c1 = 2 * p + 1
        pltpu.make_async_copy(msg_hbm.at[rows(c0)], mb0, sem0).wait()
        pltpu.sync_copy(mb0, acc.at[ivm.at[c0]], add=True)

        @pl.when(c0 + 2 < nch)
        def _():
            pltpu.async_copy(msg_hbm.at[rows(c0 + 2)], mb0, sem0)
        pltpu.make_async_copy(msg_hbm.at[rows(c1)], mb1, sem1).wait()
        pltpu.sync_copy(mb1, acc.at[ivm.at[c1]], add=True)

        @pl.when(c1 + 2 < nch)
        def _():
            pltpu.async_copy(msg_hbm.at[rows(c1 + 2)], mb1, sem1)
        return 0
    lax.fori_loop(0, nch // 2, pairbody, 0)
    if nch % 2:
        pltpu.make_async_copy(msg_hbm.at[rows(nch - 1)], mb0, sem0).wait()
        pltpu.sync_copy(mb0, acc.at[ivm.at[nch - 1]], add=True)
    plsc.subcore_barrier()

    @pl.when(sid < 15)
    def _():
        pltpu.sync_copy(acc.at[pl.ds(sid * NPTA, NPTA)],
                        part_hbm.at[cid, pl.ds(sid * NPTA, NPTA)])

    @pl.when(sid == 15)
    def _():
        pltpu.sync_copy(acc.at[pl.ds(15 * NPTA, NPTL)],
                        part_hbm.at[cid, pl.ds(15 * NPTA, NPTL)])


def _sc_scatter(msg, i2d, zrows, pt, nch):
    mesh = plsc.VectorSubcoreMesh(core_axis_name="c", subcore_axis_name="s")
    fn = pl.kernel(
        functools.partial(_scatter_body, pt, nch),
        out_type=jax.ShapeDtypeStruct((2, N_NODES, MROW), jnp.float32),
        mesh=mesh,
        scratch_types=[
            pltpu.VMEM((nch, CH), jnp.int32),
            pltpu.VMEM((CH, MROW), jnp.float32),
            pltpu.VMEM((CH, MROW), jnp.float32),
            pltpu.VMEM_SHARED((N_NODES, MROW), jnp.float32),
            pltpu.SemaphoreType.DMA,
            pltpu.SemaphoreType.DMA,
        ],
        compiler_params=_SC_PARAMS,
    )
    return fn(msg, i2d, zrows)


def _finish_body(pa, pb, wb, out):
    s = (pa[0] + pa[1] + pb[0] + pb[1]) * (1.0 / (NUM_AVG_NEIGH ** 0.5))
    out[...] = jnp.dot(s, wb[...], preferred_element_type=jnp.float32)


def _tc_finish(part_a, part_b, wb):
    return pl.pallas_call(
        _finish_body,
        out_shape=jax.ShapeDtypeStruct((N_NODES, MROW), jnp.float32),
    )(part_a, part_b, wb)


def kernel(edge_vector, edge_idx, atom_type, atom_feats_0, radial_W,
           mlp0_W0, mlp0_b0, mlp0_W1, mlp0_b1, mlp0_W2, mlp0_b2,
           mlp1_W0, mlp1_b0, mlp1_W1, mlp1_b1, mlp1_W2, mlp1_b2,
           mlp2_W0, mlp2_b0, mlp2_W1, mlp2_b1, mlp2_W2, mlp2_b2,
           chan0_W, chan1_W, chan2_W):
    f32 = jnp.float32
    i_row = edge_idx[0]
    j_row = edge_idx[1]
    ftab_flat = atom_feats_0.T.reshape(N_NODES * 8)

    tab = radial_W.reshape(16, N_U, N_CHEB).reshape(128, N_CHEB)
    w0 = jnp.concatenate([mlp0_W0, mlp1_W0, mlp2_W0], axis=1).T

    def bd(a, b, c):
        z = jnp.zeros((24, 24), f32)
        return z.at[0:8, 0:8].set(a).at[8:16, 8:16].set(b).at[16:24, 16:24].set(c)

    w1 = bd(mlp0_W1, mlp1_W1, mlp2_W1).T
    w2 = bd(mlp0_W2, mlp1_W2, mlp2_W2).T
    b0 = jnp.concatenate([mlp0_b0, mlp1_b0, mlp2_b0])[:, None]
    b1 = jnp.concatenate([mlp0_b1, mlp1_b1, mlp2_b1])[:, None]
    b2 = jnp.concatenate([mlp0_b2, mlp1_b2, mlp2_b2])[:, None]
    wb = jnp.zeros((MROW, MROW), f32)
    wb = wb.at[0:8, 0:8].set(chan0_W.T)
    wb = wb.at[8:32, 8:32].set(jnp.kron(jnp.eye(3, dtype=f32), chan1_W.T))
    wb = wb.at[32:104, 32:104].set(jnp.kron(jnp.eye(9, dtype=f32), chan2_W.T))

    zrows = jnp.zeros((N_NODES, MROW), f32)
    parts = []
    for (lo, half, pt, chunks) in ((0, HALF_A, PT_A, CHUNKS_A),
                                   (HALF_A, HALF_B, PT_B, CHUNKS_B)):
        nch = pt // CH
        ih = lax.slice_in_dim(i_row, lo, lo + half)
        jh = lax.slice_in_dim(j_row, lo, lo + half)
        evh = jnp.concatenate(
            [lax.slice_in_dim(edge_vector, lo, lo + half).T,
             jnp.zeros((5, half), f32)], axis=0)
        h0T, pair8 = _sc_gather(ih, jh, atom_type, ftab_flat, pt, chunks, half)
        msg = _tc_dense(evh, pair8, h0T, tab, w0, w1, w2, b0, b1, b2, half)
        i2d = ih.reshape(NW, nch, CH)
        parts.append(_sc_scatter(msg, i2d, zrows, pt, nch))

    out = _tc_finish(parts[0], parts[1], wb)

    M0 = out[:, 0:8].T
    M1 = out[:, 8:32].reshape(N_NODES, 3, 8).transpose(2, 0, 1)
    M2 = out[:, 32:104].reshape(N_NODES, 3, 3, 8).transpose(3, 0, 2, 1)
    return (M0, M1, M2)


# speedup vs baseline: 260.6327x; 1.0629x over previous
import functools

import jax
import jax.numpy as jnp
from jax import lax
from jax.experimental import pallas as pl
from jax.experimental.pallas import tpu as pltpu
from jax.experimental.pallas import tpu_sc as plsc

N_NODES = 10000
N_EDGES = 320000
N_U = 8
N_CHEB = 9
R_CUT = 5.0
NUM_AVG_NEIGH = 32.0

NW = 32
CH = 80
MROW = 128
BLK = 6400
NPTA = 624
NPTL = N_NODES - 15 * NPTA

PT_A = 4000
PT_B = 6000
HALF_A = NW * PT_A
HALF_B = NW * PT_B
CHUNKS_A = ((0, 1280), (1280, 1280), (2560, 1280), (3840, 160))
CHUNKS_B = ((0, 1280), (1280, 1280), (2560, 1280), (3840, 1280), (5120, 880))
PCHMAX = 1280

_SC_PARAMS = pltpu.CompilerParams(needs_layout_passes=False,
                                  use_tc_tiling_on_sc=False)


def _gather_body(pt, chunks, i_hbm, j_hbm, type_hbm, ftab_hbm, h0T_hbm,
                 pair8_hbm, type_v, ftab, ib0, ib1, jb0, jb1, sbp, sbh,
                 semi0, semi1, semj0, semj1, semp, semh):
    cid = lax.axis_index("c")
    sid = lax.axis_index("s")
    wid = sid * 2 + cid
    base = wid * pt
    ibufs = (ib0, ib1)
    jbufs = (jb0, jb1)
    semis = (semi0, semi1)
    semjs = (semj0, semj1)

    def idx_desc(k):
        off, sz = chunks[k]
        cb = base + off
        sl = k % 2
        return (
            pltpu.make_async_copy(i_hbm.at[pl.ds(cb, sz)],
                                  ibufs[sl].at[pl.ds(0, sz)], semis[sl]),
            pltpu.make_async_copy(j_hbm.at[pl.ds(cb, sz)],
                                  jbufs[sl].at[pl.ds(0, sz)], semjs[sl]),
        )

    for d in idx_desc(0):
        d.start()
    pltpu.sync_copy(type_hbm, type_v)
    pltpu.sync_copy(ftab_hbm, ftab)

    out_descs = []
    for k, (off, sz) in enumerate(chunks):
        cb = base + off
        sl = k % 2
        ibuf = ibufs[sl]
        jbuf = jbufs[sl]
        for d in idx_desc(k):
            d.wait()
        if k + 1 < len(chunks):
            for d in idx_desc(k + 1):
                d.start()

        def grp_pair(g, _):
            iv = ibuf[pl.ds(g * 16, 16)]
            jv = jbuf[pl.ds(g * 16, 16)]
            it = plsc.load_gather(type_v, [iv])
            jt = plsc.load_gather(type_v, [jv])
            pv = (it * 4 + jt).astype(jnp.float32)
            for u in range(8):
                sbp[u, pl.ds(g * 16, 16)] = pv
            return 0
        if out_descs:
            out_descs[-2].wait()
        lax.fori_loop(0, sz // 16, grp_pair, 0)
        dp = pltpu.make_async_copy(sbp.at[:, pl.ds(0, sz)],
                                   pair8_hbm.at[:, pl.ds(cb, sz)], semp)
        dp.start()

        def grp_h0(g, _):
            jv = jbuf[pl.ds(g * 16, 16)]
            j8 = jv * 8
            for u in range(8):
                hv = plsc.load_gather(ftab, [j8 + u])
                sbh[u, pl.ds(g * 16, 16)] = hv
            return 0
        if out_descs:
            out_descs[-1].wait()
        lax.fori_loop(0, sz // 16, grp_h0, 0)
        dh = pltpu.make_async_copy(sbh.at[:, pl.ds(0, sz)],
                                   h0T_hbm.at[:, pl.ds(cb, sz)], semh)
        dh.start()
        out_descs = [dp, dh]
    for d in out_descs:
        d.wait()


def _sc_gather(i_row, j_row, atom_type, ftab_flat, pt, chunks, half):
    mesh = plsc.VectorSubcoreMesh(core_axis_name="c", subcore_axis_name="s")
    fn = pl.kernel(
        functools.partial(_gather_body, pt, chunks),
        out_type=(jax.ShapeDtypeStruct((8, half), jnp.float32),
                  jax.ShapeDtypeStruct((8, half), jnp.float32)),
        mesh=mesh,
        scratch_types=[
            pltpu.VMEM((N_NODES,), jnp.int32),
            pltpu.VMEM((N_NODES * 8,), jnp.float32),
            pltpu.VMEM((PCHMAX,), jnp.int32),
            pltpu.VMEM((PCHMAX,), jnp.int32),
            pltpu.VMEM((PCHMAX,), jnp.int32),
            pltpu.VMEM((PCHMAX,), jnp.int32),
            pltpu.VMEM((8, PCHMAX), jnp.float32),
            pltpu.VMEM((8, PCHMAX), jnp.float32),
            pltpu.SemaphoreType.DMA,
            pltpu.SemaphoreType.DMA,
            pltpu.SemaphoreType.DMA,
            pltpu.SemaphoreType.DMA,
            pltpu.SemaphoreType.DMA,
            pltpu.SemaphoreType.DMA,
        ],
        compiler_params=_SC_PARAMS,
    )
    return fn(i_row, j_row, atom_type, ftab_flat)


def _dense_body(evp, pairb, h0Tb, tab, w0, w1, w2, b0, b1, b2, out):
    x = evp[0:1, :]
    y = evp[1:2, :]
    z = evp[2:3, :]
    r = jnp.sqrt(x * x + y * y + z * z)
    xc = jnp.clip(2.0 * r / R_CUT - 1.0, -1.0, 1.0)
    Ts = [jnp.ones_like(xc), xc]
    for _ in range(2, N_CHEB):
        Ts.append(2.0 * xc * Ts[-1] - Ts[-2])
    Tcm = jnp.concatenate(Ts, axis=0)
    xr = r * (1.0 / R_CUT)
    x2 = xr * xr
    x3 = x2 * xr
    x6 = x3 * x3
    env = jnp.where(xr < 1.0,
                    1.0 - 28.0 * x6 + 48.0 * x6 * xr - 21.0 * x6 * x2,
                    0.0)
    Gm = jnp.dot(tab[...], Tcm, preferred_element_type=jnp.float32)
    pair = pairb[...]
    fu8 = jnp.zeros((8, BLK), jnp.float32)
    for p in range(16):
        fu8 = fu8 + jnp.where(pair == float(p), Gm[8 * p:8 * p + 8, :], 0.0)
    fu = env * fu8
    h1 = jnp.dot(w0[...], fu, preferred_element_type=jnp.float32) + b0[...]
    h1 = h1 * (1.0 / (1.0 + jnp.exp(-h1)))
    h2 = jnp.dot(w1[...], h1, preferred_element_type=jnp.float32) + b1[...]
    h2 = h2 * (1.0 / (1.0 + jnp.exp(-h2)))
    Rm = jnp.dot(w2[...], h2, preferred_element_type=jnp.float32) + b2[...]
    h0T = h0Tb[...]
    Rh = Rm * jnp.concatenate([h0T, h0T, h0T], axis=0)
    rs = jnp.maximum(r, 1e-12)
    unit = evp[0:3, :] / rs
    uu = jnp.concatenate([unit * unit[k:k + 1, :] for k in range(3)], axis=0)
    pieces = [Rh[0:8, :]]
    for a in range(3):
        pieces.append(Rh[8:16, :] * unit[a:a + 1, :])
    for c in range(9):
        pieces.append(Rh[16:24, :] * uu[c:c + 1, :])
    pieces.append(jnp.zeros((MROW - 104, BLK), jnp.float32))
    out[...] = jnp.concatenate(pieces, axis=0).T


def _tc_dense(evp, pair8, h0T, tab, w0, w1, w2, b0, b1, b2, half):
    nblk = half // BLK
    return pl.pallas_call(
        _dense_body,
        grid=(nblk,),
        in_specs=[
            pl.BlockSpec((8, BLK), lambda i: (0, i)),
            pl.BlockSpec((8, BLK), lambda i: (0, i)),
            pl.BlockSpec((8, BLK), lambda i: (0, i)),
            pl.BlockSpec((128, N_CHEB), lambda i: (0, 0)),
            pl.BlockSpec((24, 8), lambda i: (0, 0)),
            pl.BlockSpec((24, 24), lambda i: (0, 0)),
            pl.BlockSpec((24, 24), lambda i: (0, 0)),
            pl.BlockSpec((24, 1), lambda i: (0, 0)),
            pl.BlockSpec((24, 1), lambda i: (0, 0)),
            pl.BlockSpec((24, 1), lambda i: (0, 0)),
        ],
        out_specs=pl.BlockSpec((BLK, MROW), lambda i: (i, 0)),
        out_shape=jax.ShapeDtypeStruct((half, MROW), jnp.float32),
    )(evp, pair8, h0T, tab, w0, w1, w2, b0, b1, b2)


def _scatter_body(pt, nch, msg_hbm, i2d_hbm, zrows_hbm, part_hbm, ivm, mb0,
                  mb1, acc, sem0, sem1):
    cid = lax.axis_index("c")
    sid = lax.axis_index("s")
    wid = sid * 2 + cid
    ebase = wid * pt
    pltpu.sync_copy(i2d_hbm.at[wid], ivm)

    @pl.when(sid < 15)
    def _():
        pltpu.sync_copy(zrows_hbm.at[pl.ds(sid * NPTA, NPTA)],
                        acc.at[pl.ds(sid * NPTA, NPTA)])

    @pl.when(sid == 15)
    def _():
        pltpu.sync_copy(zrows_hbm.at[pl.ds(15 * NPTA, NPTL)],
                        acc.at[pl.ds(15 * NPTA, NPTL)])
    plsc.subcore_barrier()

    def rows(c):
        return pl.ds(ebase + c * CH, CH)

    pltpu.async_copy(msg_hbm.at[rows(0)], mb0, sem0)
    pltpu.async_copy(msg_hbm.at[rows(1)], mb1, sem1)

    def pairbody(p, _):
        c0 = 2 * p
        c1 = 2 * p + 1
        pltpu.make_async_copy(msg_hbm.at[rows(c0)], mb0, sem0).wait()
        pltpu.sync_copy(mb0, acc.at[ivm.at[c0]], add=True)

        @pl.when(c0 + 2 < nch)
        def _():
            pltpu.async_copy(msg_hbm.at[rows(c0 + 2)], mb0, sem0)
        pltpu.make_async_copy(msg_hbm.at[rows(c1)], mb1, sem1).wait()
        pltpu.sync_copy(mb1, acc.at[ivm.at[c1]], add=True)

        @pl.when(c1 + 2 < nch)
        def _():
            pltpu.async_copy(msg_hbm.at[rows(c1 + 2)], mb1, sem1)
        return 0
    lax.fori_loop(0, nch // 2, pairbody, 0)
    if nch % 2:
        pltpu.make_async_copy(msg_hbm.at[rows(nch - 1)], mb0, sem0).wait()
        pltpu.sync_copy(mb0, acc.at[ivm.at[nch - 1]], add=True)
    plsc.subcore_barrier()

    @pl.when(sid < 15)
    def _():
        pltpu.sync_copy(acc.at[pl.ds(sid * NPTA, NPTA)],
                        part_hbm.at[cid, pl.ds(sid * NPTA, NPTA)])

    @pl.when(sid == 15)
    def _():
        pltpu.sync_copy(acc.at[pl.ds(15 * NPTA, NPTL)],
                        part_hbm.at[cid, pl.ds(15 * NPTA, NPTL)])


def _sc_scatter(msg, i2d, zrows, pt, nch):
    mesh = plsc.VectorSubcoreMesh(core_axis_name="c", subcore_axis_name="s")
    fn = pl.kernel(
        functools.partial(_scatter_body, pt, nch),
        out_type=jax.ShapeDtypeStruct((2, N_NODES, MROW), jnp.float32),
        mesh=mesh,
        scratch_types=[
            pltpu.VMEM((nch, CH), jnp.int32),
            pltpu.VMEM((CH, MROW), jnp.float32),
            pltpu.VMEM((CH, MROW), jnp.float32),
            pltpu.VMEM_SHARED((N_NODES, MROW), jnp.float32),
            pltpu.SemaphoreType.DMA,
            pltpu.SemaphoreType.DMA,
        ],
        compiler_params=_SC_PARAMS,
    )
    return fn(msg, i2d, zrows)


def _finish_body(pa, pb, wbT, m0, m1cm, m2cm):
    s = (pa[0] + pa[1] + pb[0] + pb[1]) * (1.0 / (NUM_AVG_NEIGH ** 0.5))
    outT = jnp.dot(wbT[...], s.T, preferred_element_type=jnp.float32)
    m0[...] = outT[0:8, :]
    m1cm[...] = outT[8:32, :]
    m2cm[...] = outT[32:104, :]


def _tc_finish(part_a, part_b, wbT):
    return pl.pallas_call(
        _finish_body,
        out_shape=(jax.ShapeDtypeStruct((8, N_NODES), jnp.float32),
                   jax.ShapeDtypeStruct((24, N_NODES), jnp.float32),
                   jax.ShapeDtypeStruct((72, N_NODES), jnp.float32)),
    )(part_a, part_b, wbT)


def kernel(edge_vector, edge_idx, atom_type, atom_feats_0, radial_W,
           mlp0_W0, mlp0_b0, mlp0_W1, mlp0_b1, mlp0_W2, mlp0_b2,
           mlp1_W0, mlp1_b0, mlp1_W1, mlp1_b1, mlp1_W2, mlp1_b2,
           mlp2_W0, mlp2_b0, mlp2_W1, mlp2_b1, mlp2_W2, mlp2_b2,
           chan0_W, chan1_W, chan2_W):
    f32 = jnp.float32
    i_row = edge_idx[0]
    j_row = edge_idx[1]
    ftab_flat = atom_feats_0.T.reshape(N_NODES * 8)

    tab = radial_W.reshape(16, N_U, N_CHEB).reshape(128, N_CHEB)
    w0 = jnp.concatenate([mlp0_W0, mlp1_W0, mlp2_W0], axis=1).T

    def bd(a, b, c):
        z = jnp.zeros((24, 24), f32)
        return z.at[0:8, 0:8].set(a).at[8:16, 8:16].set(b).at[16:24, 16:24].set(c)

    w1 = bd(mlp0_W1, mlp1_W1, mlp2_W1).T
    w2 = bd(mlp0_W2, mlp1_W2, mlp2_W2).T
    b0 = jnp.concatenate([mlp0_b0, mlp1_b0, mlp2_b0])[:, None]
    b1 = jnp.concatenate([mlp0_b1, mlp1_b1, mlp2_b1])[:, None]
    b2 = jnp.concatenate([mlp0_b2, mlp1_b2, mlp2_b2])[:, None]
    wb = jnp.zeros((MROW, MROW), f32)
    wb = wb.at[0:8, 0:8].set(chan0_W.T)
    wb = wb.at[8:32, 8:32].set(jnp.kron(jnp.eye(3, dtype=f32), chan1_W.T))
    wb = wb.at[32:104, 32:104].set(jnp.kron(jnp.eye(9, dtype=f32), chan2_W.T))

    zrows = jnp.zeros((N_NODES, MROW), f32)
    parts = []
    for (lo, half, pt, chunks) in ((0, HALF_A, PT_A, CHUNKS_A),
                                   (HALF_A, HALF_B, PT_B, CHUNKS_B)):
        nch = pt // CH
        ih = lax.slice_in_dim(i_row, lo, lo + half)
        jh = lax.slice_in_dim(j_row, lo, lo + half)
        evh = jnp.concatenate(
            [lax.slice_in_dim(edge_vector, lo, lo + half).T,
             jnp.zeros((5, half), f32)], axis=0)
        h0T, pair8 = _sc_gather(ih, jh, atom_type, ftab_flat, pt, chunks, half)
        msg = _tc_dense(evh, pair8, h0T, tab, w0, w1, w2, b0, b1, b2, half)
        i2d = ih.reshape(NW, nch, CH)
        parts.append(_sc_scatter(msg, i2d, zrows, pt, nch))

    M0, m1cm, m2cm = _tc_finish(parts[0], parts[1], wb.T)

    M1 = m1cm.reshape(3, 8, N_NODES).transpose(1, 2, 0)
    M2 = m2cm.reshape(9, 8, N_NODES).transpose(1, 2, 0).reshape(
        8, N_NODES, 3, 3)
    return (M0, M1, M2)


# speedup vs baseline: 266.3885x; 1.0221x over previous
import functools

import jax
import jax.numpy as jnp
from jax import lax
from jax.experimental import pallas as pl
from jax.experimental.pallas import tpu as pltpu
from jax.experimental.pallas import tpu_sc as plsc

N_NODES = 10000
N_EDGES = 320000
N_U = 8
N_CHEB = 9
R_CUT = 5.0
NUM_AVG_NEIGH = 32.0

NW = 32
CH = 80
MROW = 128
BLK = 6400
NPTA = 624
NPTL = N_NODES - 15 * NPTA

PT_A = 4000
PT_B = 6000
HALF_A = NW * PT_A
HALF_B = NW * PT_B
CHUNKS_A = ((0, 1280), (1280, 1280), (2560, 1280), (3840, 160))
CHUNKS_B = ((0, 1280), (1280, 1280), (2560, 1280), (3840, 1280), (5120, 880))
PCHMAX = 1280

_SC_PARAMS = pltpu.CompilerParams(needs_layout_passes=False,
                                  use_tc_tiling_on_sc=False)


def _gather_body(pt, chunks, i_hbm, j_hbm, type_hbm, ftab_hbm, h0T_hbm,
                 pair8_hbm, type_v, ftab, ib0, ib1, jb0, jb1, sbp, sbh,
                 semi0, semi1, semj0, semj1, semp, semh):
    cid = lax.axis_index("c")
    sid = lax.axis_index("s")
    wid = sid * 2 + cid
    base = wid * pt
    ibufs = (ib0, ib1)
    jbufs = (jb0, jb1)
    semis = (semi0, semi1)
    semjs = (semj0, semj1)

    def idx_desc(k):
        off, sz = chunks[k]
        cb = base + off
        sl = k % 2
        return (
            pltpu.make_async_copy(i_hbm.at[pl.ds(cb, sz)],
                                  ibufs[sl].at[pl.ds(0, sz)], semis[sl]),
            pltpu.make_async_copy(j_hbm.at[pl.ds(cb, sz)],
                                  jbufs[sl].at[pl.ds(0, sz)], semjs[sl]),
        )

    for d in idx_desc(0):
        d.start()
    pltpu.sync_copy(type_hbm, type_v)
    pltpu.sync_copy(ftab_hbm, ftab)

    out_descs = []
    for k, (off, sz) in enumerate(chunks):
        cb = base + off
        sl = k % 2
        ibuf = ibufs[sl]
        jbuf = jbufs[sl]
        for d in idx_desc(k):
            d.wait()
        if k + 1 < len(chunks):
            for d in idx_desc(k + 1):
                d.start()

        def grp_pair(g, _):
            iv = ibuf[pl.ds(g * 16, 16)]
            jv = jbuf[pl.ds(g * 16, 16)]
            it = plsc.load_gather(type_v, [iv])
            jt = plsc.load_gather(type_v, [jv])
            pv = (it * 4 + jt).astype(jnp.float32)
            for u in range(8):
                sbp[u, pl.ds(g * 16, 16)] = pv
            return 0
        if out_descs:
            out_descs[-2].wait()
        lax.fori_loop(0, sz // 16, grp_pair, 0)
        dp = pltpu.make_async_copy(sbp.at[:, pl.ds(0, sz)],
                                   pair8_hbm.at[:, pl.ds(cb, sz)], semp)
        dp.start()

        def grp_h0(g, _):
            jv = jbuf[pl.ds(g * 16, 16)]
            j8 = jv * 8
            for u in range(8):
                hv = plsc.load_gather(ftab, [j8 + u])
                sbh[u, pl.ds(g * 16, 16)] = hv
            return 0
        if out_descs:
            out_descs[-1].wait()
        lax.fori_loop(0, sz // 16, grp_h0, 0)
        dh = pltpu.make_async_copy(sbh.at[:, pl.ds(0, sz)],
                                   h0T_hbm.at[:, pl.ds(cb, sz)], semh)
        dh.start()
        out_descs = [dp, dh]
    for d in out_descs:
        d.wait()


def _sc_gather(i_row, j_row, atom_type, ftab_flat, pt, chunks, half):
    mesh = plsc.VectorSubcoreMesh(core_axis_name="c", subcore_axis_name="s")
    fn = pl.kernel(
        functools.partial(_gather_body, pt, chunks),
        out_type=(jax.ShapeDtypeStruct((8, half), jnp.float32),
                  jax.ShapeDtypeStruct((8, half), jnp.float32)),
        mesh=mesh,
        scratch_types=[
            pltpu.VMEM((N_NODES,), jnp.int32),
            pltpu.VMEM((N_NODES * 8,), jnp.float32),
            pltpu.VMEM((PCHMAX,), jnp.int32),
            pltpu.VMEM((PCHMAX,), jnp.int32),
            pltpu.VMEM((PCHMAX,), jnp.int32),
            pltpu.VMEM((PCHMAX,), jnp.int32),
            pltpu.VMEM((8, PCHMAX), jnp.float32),
            pltpu.VMEM((8, PCHMAX), jnp.float32),
            pltpu.SemaphoreType.DMA,
            pltpu.SemaphoreType.DMA,
            pltpu.SemaphoreType.DMA,
            pltpu.SemaphoreType.DMA,
            pltpu.SemaphoreType.DMA,
            pltpu.SemaphoreType.DMA,
        ],
        compiler_params=_SC_PARAMS,
    )
    return fn(i_row, j_row, atom_type, ftab_flat)


def _dense_body(evp, pairb, h0Tb, tab, w0, w1, w2, b0, b1, b2, out):
    x = evp[0:1, :]
    y = evp[1:2, :]
    z = evp[2:3, :]
    r = jnp.sqrt(x * x + y * y + z * z)
    xc = jnp.clip(2.0 * r / R_CUT - 1.0, -1.0, 1.0)
    Ts = [jnp.ones_like(xc), xc]
    for _ in range(2, N_CHEB):
        Ts.append(2.0 * xc * Ts[-1] - Ts[-2])
    Tcm = jnp.concatenate(Ts, axis=0)
    xr = r * (1.0 / R_CUT)
    x2 = xr * xr
    x3 = x2 * xr
    x6 = x3 * x3
    env = jnp.where(xr < 1.0,
                    1.0 - 28.0 * x6 + 48.0 * x6 * xr - 21.0 * x6 * x2,
                    0.0)
    Gm = jnp.dot(tab[...], Tcm, preferred_element_type=jnp.float32)
    pair = pairb[...]
    fu8 = jnp.zeros((8, BLK), jnp.float32)
    for p in range(16):
        fu8 = fu8 + jnp.where(pair == float(p), Gm[8 * p:8 * p + 8, :], 0.0)
    fu = env * fu8
    h1 = jnp.dot(w0[...], fu, preferred_element_type=jnp.float32) + b0[...]
    h1 = h1 * (1.0 / (1.0 + jnp.exp(-h1)))
    h2 = jnp.dot(w1[...], h1, preferred_element_type=jnp.float32) + b1[...]
    h2 = h2 * (1.0 / (1.0 + jnp.exp(-h2)))
    Rm = jnp.dot(w2[...], h2, preferred_element_type=jnp.float32) + b2[...]
    h0T = h0Tb[...]
    Rh = Rm * jnp.concatenate([h0T, h0T, h0T], axis=0)
    rs = jnp.maximum(r, 1e-12)
    unit = evp[0:3, :] / rs
    uu = jnp.concatenate([unit * unit[k:k + 1, :] for k in range(3)], axis=0)
    pieces = [Rh[0:8, :]]
    for a in range(3):
        pieces.append(Rh[8:16, :] * unit[a:a + 1, :])
    for c in range(9):
        pieces.append(Rh[16:24, :] * uu[c:c + 1, :])
    pieces.append(jnp.zeros((MROW - 104, BLK), jnp.float32))
    out[...] = jnp.concatenate(pieces, axis=0).T


def _tc_dense(evp, pair8, h0T, tab, w0, w1, w2, b0, b1, b2, half):
    nblk = half // BLK
    return pl.pallas_call(
        _dense_body,
        grid=(nblk,),
        in_specs=[
            pl.BlockSpec((8, BLK), lambda i: (0, i)),
            pl.BlockSpec((8, BLK), lambda i: (0, i)),
            pl.BlockSpec((8, BLK), lambda i: (0, i)),
            pl.BlockSpec((128, N_CHEB), lambda i: (0, 0)),
            pl.BlockSpec((24, 8), lambda i: (0, 0)),
            pl.BlockSpec((24, 24), lambda i: (0, 0)),
            pl.BlockSpec((24, 24), lambda i: (0, 0)),
            pl.BlockSpec((24, 1), lambda i: (0, 0)),
            pl.BlockSpec((24, 1), lambda i: (0, 0)),
            pl.BlockSpec((24, 1), lambda i: (0, 0)),
        ],
        out_specs=pl.BlockSpec((BLK, MROW), lambda i: (i, 0)),
        out_shape=jax.ShapeDtypeStruct((half, MROW), jnp.float32),
    )(evp, pair8, h0T, tab, w0, w1, w2, b0, b1, b2)


def _scatter_body(pt, nch, msg_hbm, i2d_hbm, zrows_hbm, part_hbm, ivm, mb0,
                  mb1, mb2, mb3, acc, semr0, semr1, semr2, semr3,
                  sema0, sema1, sema2, sema3):
    cid = lax.axis_index("c")
    sid = lax.axis_index("s")
    wid = sid * 2 + cid
    ebase = wid * pt
    pltpu.sync_copy(i2d_hbm.at[wid], ivm)

    @pl.when(sid < 15)
    def _():
        pltpu.sync_copy(zrows_hbm.at[pl.ds(sid * NPTA, NPTA)],
                        acc.at[pl.ds(sid * NPTA, NPTA)])

    @pl.when(sid == 15)
    def _():
        pltpu.sync_copy(zrows_hbm.at[pl.ds(15 * NPTA, NPTL)],
                        acc.at[pl.ds(15 * NPTA, NPTL)])
    plsc.subcore_barrier()

    def rows(c):
        return pl.ds(ebase + c * CH, CH)

    mbs = (mb0, mb1, mb2, mb3)
    semr = (semr0, semr1, semr2, semr3)
    sema = (sema0, sema1, sema2, sema3)
    for b in range(4):
        pltpu.async_copy(msg_hbm.at[rows(b)], mbs[b], semr[b])

    def group(q, _):
        adds = []
        for b in range(4):
            c = 4 * q + b
            pltpu.make_async_copy(msg_hbm.at[rows(c)], mbs[b], semr[b]).wait()
            adds.append(pltpu.async_copy(mbs[b], acc.at[ivm.at[c]], sema[b],
                                         add=True))
        for b in range(4):
            c = 4 * q + b
            adds[b].wait()

            @pl.when(c + 4 < nch)
            def _():
                pltpu.async_copy(msg_hbm.at[rows(c + 4)], mbs[b], semr[b])
        return 0
    lax.fori_loop(0, nch // 4, group, 0)
    for c in range(nch - nch % 4, nch):
        b = c % 4
        pltpu.make_async_copy(msg_hbm.at[rows(c)], mbs[b], semr[b]).wait()
        pltpu.sync_copy(mbs[b], acc.at[ivm.at[c]], add=True)
    plsc.subcore_barrier()

    @pl.when(sid < 15)
    def _():
        pltpu.sync_copy(acc.at[pl.ds(sid * NPTA, NPTA)],
                        part_hbm.at[cid, pl.ds(sid * NPTA, NPTA)])

    @pl.when(sid == 15)
    def _():
        pltpu.sync_copy(acc.at[pl.ds(15 * NPTA, NPTL)],
                        part_hbm.at[cid, pl.ds(15 * NPTA, NPTL)])


def _sc_scatter(msg, i2d, zrows, pt, nch):
    mesh = plsc.VectorSubcoreMesh(core_axis_name="c", subcore_axis_name="s")
    fn = pl.kernel(
        functools.partial(_scatter_body, pt, nch),
        out_type=jax.ShapeDtypeStruct((2, N_NODES, MROW), jnp.float32),
        mesh=mesh,
        scratch_types=[
            pltpu.VMEM((nch, CH), jnp.int32),
            pltpu.VMEM((CH, MROW), jnp.float32),
            pltpu.VMEM((CH, MROW), jnp.float32),
            pltpu.VMEM((CH, MROW), jnp.float32),
            pltpu.VMEM((CH, MROW), jnp.float32),
            pltpu.VMEM_SHARED((N_NODES, MROW), jnp.float32),
            pltpu.SemaphoreType.DMA,
            pltpu.SemaphoreType.DMA,
            pltpu.SemaphoreType.DMA,
            pltpu.SemaphoreType.DMA,
            pltpu.SemaphoreType.DMA,
            pltpu.SemaphoreType.DMA,
            pltpu.SemaphoreType.DMA,
            pltpu.SemaphoreType.DMA,
        ],
        compiler_params=_SC_PARAMS,
    )
    return fn(msg, i2d, zrows)


def _finish_body(pa, pb, wbT, m0, m1cm, m2cm):
    s = (pa[0] + pa[1] + pb[0] + pb[1]) * (1.0 / (NUM_AVG_NEIGH ** 0.5))
    outT = jnp.dot(wbT[...], s.T, preferred_element_type=jnp.float32)
    m0[...] = outT[0:8, :]
    m1cm[...] = outT[8:32, :]
    m2cm[...] = outT[32:104, :]


def _tc_finish(part_a, part_b, wbT):
    return pl.pallas_call(
        _finish_body,
        out_shape=(jax.ShapeDtypeStruct((8, N_NODES), jnp.float32),
                   jax.ShapeDtypeStruct((24, N_NODES), jnp.float32),
                   jax.ShapeDtypeStruct((72, N_NODES), jnp.float32)),
    )(part_a, part_b, wbT)


def kernel(edge_vector, edge_idx, atom_type, atom_feats_0, radial_W,
           mlp0_W0, mlp0_b0, mlp0_W1, mlp0_b1, mlp0_W2, mlp0_b2,
           mlp1_W0, mlp1_b0, mlp1_W1, mlp1_b1, mlp1_W2, mlp1_b2,
           mlp2_W0, mlp2_b0, mlp2_W1, mlp2_b1, mlp2_W2, mlp2_b2,
           chan0_W, chan1_W, chan2_W):
    f32 = jnp.float32
    i_row = edge_idx[0]
    j_row = edge_idx[1]
    ftab_flat = atom_feats_0.T.reshape(N_NODES * 8)

    tab = radial_W.reshape(16, N_U, N_CHEB).reshape(128, N_CHEB)
    w0 = jnp.concatenate([mlp0_W0, mlp1_W0, mlp2_W0], axis=1).T

    def bd(a, b, c):
        z = jnp.zeros((24, 24), f32)
        return z.at[0:8, 0:8].set(a).at[8:16, 8:16].set(b).at[16:24, 16:24].set(c)

    w1 = bd(mlp0_W1, mlp1_W1, mlp2_W1).T
    w2 = bd(mlp0_W2, mlp1_W2, mlp2_W2).T
    b0 = jnp.concatenate([mlp0_b0, mlp1_b0, mlp2_b0])[:, None]
    b1 = jnp.concatenate([mlp0_b1, mlp1_b1, mlp2_b1])[:, None]
    b2 = jnp.concatenate([mlp0_b2, mlp1_b2, mlp2_b2])[:, None]
    wb = jnp.zeros((MROW, MROW), f32)
    wb = wb.at[0:8, 0:8].set(chan0_W.T)
    wb = wb.at[8:32, 8:32].set(jnp.kron(jnp.eye(3, dtype=f32), chan1_W.T))
    wb = wb.at[32:104, 32:104].set(jnp.kron(jnp.eye(9, dtype=f32), chan2_W.T))

    zrows = jnp.zeros((N_NODES, MROW), f32)
    parts = []
    for (lo, half, pt, chunks) in ((0, HALF_A, PT_A, CHUNKS_A),
                                   (HALF_A, HALF_B, PT_B, CHUNKS_B)):
        nch = pt // CH
        ih = lax.slice_in_dim(i_row, lo, lo + half)
        jh = lax.slice_in_dim(j_row, lo, lo + half)
        evh = jnp.concatenate(
            [lax.slice_in_dim(edge_vector, lo, lo + half).T,
             jnp.zeros((5, half), f32)], axis=0)
        h0T, pair8 = _sc_gather(ih, jh, atom_type, ftab_flat, pt, chunks, half)
        msg = _tc_dense(evh, pair8, h0T, tab, w0, w1, w2, b0, b1, b2, half)
        i2d = ih.reshape(NW, nch, CH)
        parts.append(_sc_scatter(msg, i2d, zrows, pt, nch))

    M0, m1cm, m2cm = _tc_finish(parts[0], parts[1], wb.T)

    M1 = m1cm.reshape(3, 8, N_NODES).transpose(1, 2, 0)
    M2 = m2cm.reshape(9, 8, N_NODES).transpose(1, 2, 0).reshape(
        8, N_NODES, 3, 3)
    return (M0, M1, M2)


# speedup vs baseline: 266.7897x; 1.0015x over previous
import functools

import jax
import jax.numpy as jnp
from jax import lax
from jax.experimental import pallas as pl
from jax.experimental.pallas import tpu as pltpu
from jax.experimental.pallas import tpu_sc as plsc

N_NODES = 10000
N_EDGES = 320000
N_U = 8
N_CHEB = 9
R_CUT = 5.0
NUM_AVG_NEIGH = 32.0

NW = 32
CH = 80
MROW = 128
BLK = 6400
NPTA = 624
NPTL = N_NODES - 15 * NPTA

PT_A = 4000
PT_B = 6000
HALF_A = NW * PT_A
HALF_B = NW * PT_B
CHUNKS_A = ((0, 1280), (1280, 1280), (2560, 1280), (3840, 160))
CHUNKS_B = ((0, 1280), (1280, 1280), (2560, 1280), (3840, 1280), (5120, 880))
PCHMAX = 1280

_SC_PARAMS = pltpu.CompilerParams(needs_layout_passes=False,
                                  use_tc_tiling_on_sc=False)


def _gather_body(pt, chunks, i_hbm, j_hbm, type_hbm, ftab_hbm, h0T_hbm,
                 pair8_hbm, type_v, ftab, ib0, ib1, jb0, jb1, sbp, sbh,
                 semi0, semi1, semj0, semj1, semp, semh):
    cid = lax.axis_index("c")
    sid = lax.axis_index("s")
    wid = sid * 2 + cid
    base = wid * pt
    ibufs = (ib0, ib1)
    jbufs = (jb0, jb1)
    semis = (semi0, semi1)
    semjs = (semj0, semj1)

    def idx_desc(k):
        off, sz = chunks[k]
        cb = base + off
        sl = k % 2
        return (
            pltpu.make_async_copy(i_hbm.at[pl.ds(cb, sz)],
                                  ibufs[sl].at[pl.ds(0, sz)], semis[sl]),
            pltpu.make_async_copy(j_hbm.at[pl.ds(cb, sz)],
                                  jbufs[sl].at[pl.ds(0, sz)], semjs[sl]),
        )

    for d in idx_desc(0):
        d.start()
    pltpu.sync_copy(type_hbm, type_v)
    pltpu.sync_copy(ftab_hbm, ftab)

    out_descs = []
    for k, (off, sz) in enumerate(chunks):
        cb = base + off
        sl = k % 2
        ibuf = ibufs[sl]
        jbuf = jbufs[sl]
        for d in idx_desc(k):
            d.wait()
        if k + 1 < len(chunks):
            for d in idx_desc(k + 1):
                d.start()

        def grp_pair(g, _):
            iv = ibuf[pl.ds(g * 16, 16)]
            jv = jbuf[pl.ds(g * 16, 16)]
            it = plsc.load_gather(type_v, [iv])
            jt = plsc.load_gather(type_v, [jv])
            pv = (it * 4 + jt).astype(jnp.float32)
            for u in range(8):
                sbp[u, pl.ds(g * 16, 16)] = pv
            return 0
        if out_descs:
            out_descs[-2].wait()
        lax.fori_loop(0, sz // 16, grp_pair, 0)
        dp = pltpu.make_async_copy(sbp.at[:, pl.ds(0, sz)],
                                   pair8_hbm.at[:, pl.ds(cb, sz)], semp)
        dp.start()

        def grp_h0(g, _):
            jv = jbuf[pl.ds(g * 16, 16)]
            j8 = jv * 8
            for u in range(8):
                hv = plsc.load_gather(ftab, [j8 + u])
                sbh[u, pl.ds(g * 16, 16)] = hv
            return 0
        if out_descs:
            out_descs[-1].wait()
        lax.fori_loop(0, sz // 16, grp_h0, 0)
        dh = pltpu.make_async_copy(sbh.at[:, pl.ds(0, sz)],
                                   h0T_hbm.at[:, pl.ds(cb, sz)], semh)
        dh.start()
        out_descs = [dp, dh]
    for d in out_descs:
        d.wait()


def _sc_gather(i_row, j_row, atom_type, ftab_flat, pt, chunks, half):
    mesh = plsc.VectorSubcoreMesh(core_axis_name="c", subcore_axis_name="s")
    fn = pl.kernel(
        functools.partial(_gather_body, pt, chunks),
        out_type=(jax.ShapeDtypeStruct((8, half), jnp.float32),
                  jax.ShapeDtypeStruct((8, half), jnp.float32)),
        mesh=mesh,
        scratch_types=[
            pltpu.VMEM((N_NODES,), jnp.int32),
            pltpu.VMEM((N_NODES * 8,), jnp.float32),
            pltpu.VMEM((PCHMAX,), jnp.int32),
            pltpu.VMEM((PCHMAX,), jnp.int32),
            pltpu.VMEM((PCHMAX,), jnp.int32),
            pltpu.VMEM((PCHMAX,), jnp.int32),
            pltpu.VMEM((8, PCHMAX), jnp.float32),
            pltpu.VMEM((8, PCHMAX), jnp.float32),
            pltpu.SemaphoreType.DMA,
            pltpu.SemaphoreType.DMA,
            pltpu.SemaphoreType.DMA,
            pltpu.SemaphoreType.DMA,
            pltpu.SemaphoreType.DMA,
            pltpu.SemaphoreType.DMA,
        ],
        compiler_params=_SC_PARAMS,
    )
    return fn(i_row, j_row, atom_type, ftab_flat)


def _dense_body(evp, pairb, h0Tb, tab, w0, w1, w2, b0, b1, b2, out):
    x = evp[0:1, :]
    y = evp[1:2, :]
    z = evp[2:3, :]
    r = jnp.sqrt(x * x + y * y + z * z)
    xc = jnp.clip(2.0 * r / R_CUT - 1.0, -1.0, 1.0)
    Ts = [jnp.ones_like(xc), xc]
    for _ in range(2, N_CHEB):
        Ts.append(2.0 * xc * Ts[-1] - Ts[-2])
    Tcm = jnp.concatenate(Ts, axis=0)
    xr = r * (1.0 / R_CUT)
    x2 = xr * xr
    x3 = x2 * xr
    x6 = x3 * x3
    env = jnp.where(xr < 1.0,
                    1.0 - 28.0 * x6 + 48.0 * x6 * xr - 21.0 * x6 * x2,
                    0.0)
    Gm = jnp.dot(tab[...], Tcm, preferred_element_type=jnp.float32)
    pair = pairb[...]
    fu8 = jnp.zeros((8, BLK), jnp.float32)
    for p in range(16):
        fu8 = fu8 + jnp.where(pair == float(p), Gm[8 * p:8 * p + 8, :], 0.0)
    fu = env * fu8
    h1 = jnp.dot(w0[...], fu, preferred_element_type=jnp.float32) + b0[...]
    h1 = h1 * (1.0 / (1.0 + jnp.exp(-h1)))
    h2 = jnp.dot(w1[...], h1, preferred_element_type=jnp.float32) + b1[...]
    h2 = h2 * (1.0 / (1.0 + jnp.exp(-h2)))
    Rm = jnp.dot(w2[...], h2, preferred_element_type=jnp.float32) + b2[...]
    h0T = h0Tb[...]
    Rh = Rm * jnp.concatenate([h0T, h0T, h0T], axis=0)
    rs = jnp.maximum(r, 1e-12)
    unit = evp[0:3, :] / rs
    uu = jnp.concatenate([unit * unit[k:k + 1, :] for k in range(3)], axis=0)
    pieces = [Rh[0:8, :]]
    for a in range(3):
        pieces.append(Rh[8:16, :] * unit[a:a + 1, :])
    for c in range(9):
        pieces.append(Rh[16:24, :] * uu[c:c + 1, :])
    pieces.append(jnp.zeros((MROW - 104, BLK), jnp.float32))
    out[...] = jnp.concatenate(pieces, axis=0).T


def _tc_dense(evp, pair8, h0T, tab, w0, w1, w2, b0, b1, b2, half):
    nblk = half // BLK
    return pl.pallas_call(
        _dense_body,
        grid=(nblk,),
        in_specs=[
            pl.BlockSpec((8, BLK), lambda i: (0, i)),
            pl.BlockSpec((8, BLK), lambda i: (0, i)),
            pl.BlockSpec((8, BLK), lambda i: (0, i)),
            pl.BlockSpec((128, N_CHEB), lambda i: (0, 0)),
            pl.BlockSpec((24, 8), lambda i: (0, 0)),
            pl.BlockSpec((24, 24), lambda i: (0, 0)),
            pl.BlockSpec((24, 24), lambda i: (0, 0)),
            pl.BlockSpec((24, 1), lambda i: (0, 0)),
            pl.BlockSpec((24, 1), lambda i: (0, 0)),
            pl.BlockSpec((24, 1), lambda i: (0, 0)),
        ],
        out_specs=pl.BlockSpec((BLK, MROW), lambda i: (i, 0)),
        out_shape=jax.ShapeDtypeStruct((half, MROW), jnp.float32),
    )(evp, pair8, h0T, tab, w0, w1, w2, b0, b1, b2)


def _scatter_body(pt, nch, msg_hbm, i2d_hbm, zrows_hbm, part_hbm, ivm, mb0,
                  mb1, mb2, mb3, acc,
                  semr0, semr1, semr2, semr3,
                  sema0, sema1, sema2, sema3):
    cid = lax.axis_index("c")
    sid = lax.axis_index("s")
    wid = sid * 2 + cid
    ebase = wid * pt
    pltpu.sync_copy(i2d_hbm.at[wid], ivm)

    @pl.when(sid < 15)
    def _():
        pltpu.sync_copy(zrows_hbm.at[pl.ds(sid * NPTA, NPTA)],
                        acc.at[pl.ds(sid * NPTA, NPTA)])

    @pl.when(sid == 15)
    def _():
        pltpu.sync_copy(zrows_hbm.at[pl.ds(15 * NPTA, NPTL)],
                        acc.at[pl.ds(15 * NPTA, NPTL)])
    plsc.subcore_barrier()

    def rows(c):
        return pl.ds(ebase + c * CH, CH)

    D = 4
    mbs = (mb0, mb1, mb2, mb3)
    semr = (semr0, semr1, semr2, semr3)
    sema = (sema0, sema1, sema2, sema3)
    for b in range(D):
        pltpu.async_copy(msg_hbm.at[rows(b)], mbs[b], semr[b])

    def group(q, _):
        adds = []
        for b in range(D):
            c = D * q + b
            pltpu.make_async_copy(msg_hbm.at[rows(c)], mbs[b], semr[b]).wait()
            adds.append(pltpu.async_copy(mbs[b], acc.at[ivm.at[c]], sema[b],
                                         add=True))
        for b in range(D):
            c = D * q + b
            adds[b].wait()

            @pl.when(c + D < nch)
            def _():
                pltpu.async_copy(msg_hbm.at[rows(c + D)], mbs[b], semr[b])
        return 0
    lax.fori_loop(0, nch // D, group, 0)
    for c in range(nch - nch % D, nch):
        b = c % D
        pltpu.make_async_copy(msg_hbm.at[rows(c)], mbs[b], semr[b]).wait()
        pltpu.sync_copy(mbs[b], acc.at[ivm.at[c]], add=True)
    plsc.subcore_barrier()

    @pl.when(sid < 15)
    def _():
        pltpu.sync_copy(acc.at[pl.ds(sid * NPTA, NPTA)],
                        part_hbm.at[cid, pl.ds(sid * NPTA, NPTA)])

    @pl.when(sid == 15)
    def _():
        pltpu.sync_copy(acc.at[pl.ds(15 * NPTA, NPTL)],
                        part_hbm.at[cid, pl.ds(15 * NPTA, NPTL)])


def _sc_scatter(msg, i2d, zrows, pt, nch):
    mesh = plsc.VectorSubcoreMesh(core_axis_name="c", subcore_axis_name="s")
    fn = pl.kernel(
        functools.partial(_scatter_body, pt, nch),
        out_type=jax.ShapeDtypeStruct((2, N_NODES, MROW), jnp.float32),
        mesh=mesh,
        scratch_types=[
            pltpu.VMEM((nch, CH), jnp.int32),
            pltpu.VMEM((CH, MROW), jnp.float32),
            pltpu.VMEM((CH, MROW), jnp.float32),
            pltpu.VMEM((CH, MROW), jnp.float32),
            pltpu.VMEM((CH, MROW), jnp.float32),
            pltpu.VMEM_SHARED((N_NODES, MROW), jnp.float32),
            pltpu.SemaphoreType.DMA,
            pltpu.SemaphoreType.DMA,
            pltpu.SemaphoreType.DMA,
            pltpu.SemaphoreType.DMA,
            pltpu.SemaphoreType.DMA,
            pltpu.SemaphoreType.DMA,
            pltpu.SemaphoreType.DMA,
            pltpu.SemaphoreType.DMA,
        ],
        compiler_params=_SC_PARAMS,
    )
    return fn(msg, i2d, zrows)


def _finish_body(pa, pb, wbT, m0, m1cm, m2cm):
    s = (pa[0] + pa[1] + pb[0] + pb[1]) * (1.0 / (NUM_AVG_NEIGH ** 0.5))
    outT = jnp.dot(wbT[...], s.T, preferred_element_type=jnp.float32)
    m0[...] = outT[0:8, :]
    m1cm[...] = outT[8:32, :]
    m2cm[...] = outT[32:104, :]


def _tc_finish(part_a, part_b, wbT):
    return pl.pallas_call(
        _finish_body,
        out_shape=(jax.ShapeDtypeStruct((8, N_NODES), jnp.float32),
                   jax.ShapeDtypeStruct((24, N_NODES), jnp.float32),
                   jax.ShapeDtypeStruct((72, N_NODES), jnp.float32)),
    )(part_a, part_b, wbT)


def kernel(edge_vector, edge_idx, atom_type, atom_feats_0, radial_W,
           mlp0_W0, mlp0_b0, mlp0_W1, mlp0_b1, mlp0_W2, mlp0_b2,
           mlp1_W0, mlp1_b0, mlp1_W1, mlp1_b1, mlp1_W2, mlp1_b2,
           mlp2_W0, mlp2_b0, mlp2_W1, mlp2_b1, mlp2_W2, mlp2_b2,
           chan0_W, chan1_W, chan2_W):
    f32 = jnp.float32
    i_row = edge_idx[0]
    j_row = edge_idx[1]
    ftab_flat = atom_feats_0.T.reshape(N_NODES * 8)

    tab = radial_W.reshape(16, N_U, N_CHEB).reshape(128, N_CHEB)
    w0 = jnp.concatenate([mlp0_W0, mlp1_W0, mlp2_W0], axis=1).T

    def bd(a, b, c):
        z = jnp.zeros((24, 24), f32)
        return z.at[0:8, 0:8].set(a).at[8:16, 8:16].set(b).at[16:24, 16:24].set(c)

    w1 = bd(mlp0_W1, mlp1_W1, mlp2_W1).T
    w2 = bd(mlp0_W2, mlp1_W2, mlp2_W2).T
    b0 = jnp.concatenate([mlp0_b0, mlp1_b0, mlp2_b0])[:, None]
    b1 = jnp.concatenate([mlp0_b1, mlp1_b1, mlp2_b1])[:, None]
    b2 = jnp.concatenate([mlp0_b2, mlp1_b2, mlp2_b2])[:, None]
    wb = jnp.zeros((MROW, MROW), f32)
    wb = wb.at[0:8, 0:8].set(chan0_W.T)
    wb = wb.at[8:32, 8:32].set(jnp.kron(jnp.eye(3, dtype=f32), chan1_W.T))
    wb = wb.at[32:104, 32:104].set(jnp.kron(jnp.eye(9, dtype=f32), chan2_W.T))

    zrows = jnp.zeros((N_NODES, MROW), f32)
    parts = []
    for (lo, half, pt, chunks) in ((0, HALF_A, PT_A, CHUNKS_A),
                                   (HALF_A, HALF_B, PT_B, CHUNKS_B)):
        nch = pt // CH
        ih = lax.slice_in_dim(i_row, lo, lo + half)
        jh = lax.slice_in_dim(j_row, lo, lo + half)
        evh = jnp.concatenate(
            [lax.slice_in_dim(edge_vector, lo, lo + half).T,
             jnp.zeros((5, half), f32)], axis=0)
        h0T, pair8 = _sc_gather(ih, jh, atom_type, ftab_flat, pt, chunks, half)
        msg = _tc_dense(evh, pair8, h0T, tab, w0, w1, w2, b0, b1, b2, half)
        i2d = ih.reshape(NW, nch, CH)
        parts.append(_sc_scatter(msg, i2d, zrows, pt, nch))

    M0, m1cm, m2cm = _tc_finish(parts[0], parts[1], wb.T)

    M1 = m1cm.reshape(3, 8, N_NODES).transpose(1, 2, 0)
    M2 = m2cm.reshape(9, 8, N_NODES).transpose(1, 2, 0).reshape(
        8, N_NODES, 3, 3)
    return (M0, M1, M2)


# speedup vs baseline: 267.0259x; 1.0009x over previous
import functools

import jax
import jax.numpy as jnp
from jax import lax
from jax.experimental import pallas as pl
from jax.experimental.pallas import tpu as pltpu
from jax.experimental.pallas import tpu_sc as plsc

N_NODES = 10000
N_EDGES = 320000
N_U = 8
N_CHEB = 9
R_CUT = 5.0
NUM_AVG_NEIGH = 32.0

NW = 32
CH = 80
MROW = 128
BLK = 12800
NPTA = 624
NPTL = N_NODES - 15 * NPTA

PT_A = 4000
PT_B = 6000
HALF_A = NW * PT_A
HALF_B = NW * PT_B
CHUNKS_A = ((0, 1280), (1280, 1280), (2560, 1280), (3840, 160))
CHUNKS_B = ((0, 1280), (1280, 1280), (2560, 1280), (3840, 1280), (5120, 880))
PCHMAX = 1280

_SC_PARAMS = pltpu.CompilerParams(needs_layout_passes=False,
                                  use_tc_tiling_on_sc=False)


def _gather_body(pt, chunks, i_hbm, j_hbm, type_hbm, ftab_hbm, h0T_hbm,
                 pair8_hbm, type_v, ftab, ib0, ib1, jb0, jb1, sbp, sbh,
                 semi0, semi1, semj0, semj1, semp, semh):
    cid = lax.axis_index("c")
    sid = lax.axis_index("s")
    wid = sid * 2 + cid
    base = wid * pt
    ibufs = (ib0, ib1)
    jbufs = (jb0, jb1)
    semis = (semi0, semi1)
    semjs = (semj0, semj1)

    def idx_desc(k):
        off, sz = chunks[k]
        cb = base + off
        sl = k % 2
        return (
            pltpu.make_async_copy(i_hbm.at[pl.ds(cb, sz)],
                                  ibufs[sl].at[pl.ds(0, sz)], semis[sl]),
            pltpu.make_async_copy(j_hbm.at[pl.ds(cb, sz)],
                                  jbufs[sl].at[pl.ds(0, sz)], semjs[sl]),
        )

    for d in idx_desc(0):
        d.start()
    pltpu.sync_copy(type_hbm, type_v)
    pltpu.sync_copy(ftab_hbm, ftab)

    out_descs = []
    for k, (off, sz) in enumerate(chunks):
        cb = base + off
        sl = k % 2
        ibuf = ibufs[sl]
        jbuf = jbufs[sl]
        for d in idx_desc(k):
            d.wait()
        if k + 1 < len(chunks):
            for d in idx_desc(k + 1):
                d.start()

        def grp_pair(g, _):
            iv = ibuf[pl.ds(g * 16, 16)]
            jv = jbuf[pl.ds(g * 16, 16)]
            it = plsc.load_gather(type_v, [iv])
            jt = plsc.load_gather(type_v, [jv])
            pv = (it * 4 + jt).astype(jnp.float32)
            for u in range(8):
                sbp[u, pl.ds(g * 16, 16)] = pv
            return 0
        if out_descs:
            out_descs[-2].wait()
        lax.fori_loop(0, sz // 16, grp_pair, 0)
        dp = pltpu.make_async_copy(sbp.at[:, pl.ds(0, sz)],
                                   pair8_hbm.at[:, pl.ds(cb, sz)], semp)
        dp.start()

        def grp_h0(g, _):
            jv = jbuf[pl.ds(g * 16, 16)]
            j8 = jv * 8
            for u in range(8):
                hv = plsc.load_gather(ftab, [j8 + u])
                sbh[u, pl.ds(g * 16, 16)] = hv
            return 0
        if out_descs:
            out_descs[-1].wait()
        lax.fori_loop(0, sz // 16, grp_h0, 0)
        dh = pltpu.make_async_copy(sbh.at[:, pl.ds(0, sz)],
                                   h0T_hbm.at[:, pl.ds(cb, sz)], semh)
        dh.start()
        out_descs = [dp, dh]
    for d in out_descs:
        d.wait()


def _sc_gather(i_row, j_row, atom_type, ftab_flat, pt, chunks, half):
    mesh = plsc.VectorSubcoreMesh(core_axis_name="c", subcore_axis_name="s")
    fn = pl.kernel(
        functools.partial(_gather_body, pt, chunks),
        out_type=(jax.ShapeDtypeStruct((8, half), jnp.float32),
                  jax.ShapeDtypeStruct((8, half), jnp.float32)),
        mesh=mesh,
        scratch_types=[
            pltpu.VMEM((N_NODES,), jnp.int32),
            pltpu.VMEM((N_NODES * 8,), jnp.float32),
            pltpu.VMEM((PCHMAX,), jnp.int32),
            pltpu.VMEM((PCHMAX,), jnp.int32),
            pltpu.VMEM((PCHMAX,), jnp.int32),
            pltpu.VMEM((PCHMAX,), jnp.int32),
            pltpu.VMEM((8, PCHMAX), jnp.float32),
            pltpu.VMEM((8, PCHMAX), jnp.float32),
            pltpu.SemaphoreType.DMA,
            pltpu.SemaphoreType.DMA,
            pltpu.SemaphoreType.DMA,
            pltpu.SemaphoreType.DMA,
            pltpu.SemaphoreType.DMA,
            pltpu.SemaphoreType.DMA,
        ],
        compiler_params=_SC_PARAMS,
    )
    return fn(i_row, j_row, atom_type, ftab_flat)


def _dense_body(evp, pairb, h0Tb, tab, w0, w1, w2, b0, b1, b2, out):
    x = evp[0:1, :]
    y = evp[1:2, :]
    z = evp[2:3, :]
    r = jnp.sqrt(x * x + y * y + z * z)
    xc = jnp.clip(2.0 * r / R_CUT - 1.0, -1.0, 1.0)
    Ts = [jnp.ones_like(xc), xc]
    for _ in range(2, N_CHEB):
        Ts.append(2.0 * xc * Ts[-1] - Ts[-2])
    Tcm = jnp.concatenate(Ts, axis=0)
    xr = r * (1.0 / R_CUT)
    x2 = xr * xr
    x3 = x2 * xr
    x6 = x3 * x3
    env = jnp.where(xr < 1.0,
                    1.0 - 28.0 * x6 + 48.0 * x6 * xr - 21.0 * x6 * x2,
                    0.0)
    Gm = jnp.dot(tab[...], Tcm, preferred_element_type=jnp.float32)
    pair = pairb[...]
    fu8 = jnp.zeros((8, BLK), jnp.float32)
    for p in range(16):
        fu8 = fu8 + jnp.where(pair == float(p), Gm[8 * p:8 * p + 8, :], 0.0)
    fu = env * fu8
    h1 = jnp.dot(w0[...], fu, preferred_element_type=jnp.float32) + b0[...]
    h1 = h1 * (1.0 / (1.0 + jnp.exp(-h1)))
    h2 = jnp.dot(w1[...], h1, preferred_element_type=jnp.float32) + b1[...]
    h2 = h2 * (1.0 / (1.0 + jnp.exp(-h2)))
    Rm = jnp.dot(w2[...], h2, preferred_element_type=jnp.float32) + b2[...]
    h0T = h0Tb[...]
    Rh = Rm * jnp.concatenate([h0T, h0T, h0T], axis=0)
    rs = jnp.maximum(r, 1e-12)
    unit = evp[0:3, :] / rs
    uu = jnp.concatenate([unit * unit[k:k + 1, :] for k in range(3)], axis=0)
    pieces = [Rh[0:8, :]]
    for a in range(3):
        pieces.append(Rh[8:16, :] * unit[a:a + 1, :])
    for c in range(9):
        pieces.append(Rh[16:24, :] * uu[c:c + 1, :])
    pieces.append(jnp.zeros((MROW - 104, BLK), jnp.float32))
    out[...] = jnp.concatenate(pieces, axis=0).T


def _tc_dense(evp, pair8, h0T, tab, w0, w1, w2, b0, b1, b2, half):
    nblk = half // BLK
    return pl.pallas_call(
        _dense_body,
        grid=(nblk,),
        in_specs=[
            pl.BlockSpec((8, BLK), lambda i: (0, i)),
            pl.BlockSpec((8, BLK), lambda i: (0, i)),
            pl.BlockSpec((8, BLK), lambda i: (0, i)),
            pl.BlockSpec((128, N_CHEB), lambda i: (0, 0)),
            pl.BlockSpec((24, 8), lambda i: (0, 0)),
            pl.BlockSpec((24, 24), lambda i: (0, 0)),
            pl.BlockSpec((24, 24), lambda i: (0, 0)),
            pl.BlockSpec((24, 1), lambda i: (0, 0)),
            pl.BlockSpec((24, 1), lambda i: (0, 0)),
            pl.BlockSpec((24, 1), lambda i: (0, 0)),
        ],
        out_specs=pl.BlockSpec((BLK, MROW), lambda i: (i, 0)),
        out_shape=jax.ShapeDtypeStruct((half, MROW), jnp.float32),
    )(evp, pair8, h0T, tab, w0, w1, w2, b0, b1, b2)


def _scatter_body(pt, nch, msg_hbm, i2d_hbm, zrows_hbm, part_hbm, ivm, mb0,
                  mb1, mb2, mb3, acc,
                  semr0, semr1, semr2, semr3,
                  sema0, sema1, sema2, sema3):
    cid = lax.axis_index("c")
    sid = lax.axis_index("s")
    wid = sid * 2 + cid
    ebase = wid * pt
    pltpu.sync_copy(i2d_hbm.at[wid], ivm)

    @pl.when(sid < 15)
    def _():
        pltpu.sync_copy(zrows_hbm.at[pl.ds(sid * NPTA, NPTA)],
                        acc.at[pl.ds(sid * NPTA, NPTA)])

    @pl.when(sid == 15)
    def _():
        pltpu.sync_copy(zrows_hbm.at[pl.ds(15 * NPTA, NPTL)],
                        acc.at[pl.ds(15 * NPTA, NPTL)])
    plsc.subcore_barrier()

    def rows(c):
        return pl.ds(ebase + c * CH, CH)

    D = 4
    mbs = (mb0, mb1, mb2, mb3)
    semr = (semr0, semr1, semr2, semr3)
    sema = (sema0, sema1, sema2, sema3)
    for b in range(D):
        pltpu.async_copy(msg_hbm.at[rows(b)], mbs[b], semr[b])

    def group(q, _):
        adds = []
        for b in range(D):
            c = D * q + b
            pltpu.make_async_copy(msg_hbm.at[rows(c)], mbs[b], semr[b]).wait()
            adds.append(pltpu.async_copy(mbs[b], acc.at[ivm.at[c]], sema[b],
                                         add=True))
        for b in range(D):
            c = D * q + b
            adds[b].wait()

            @pl.when(c + D < nch)
            def _():
                pltpu.async_copy(msg_hbm.at[rows(c + D)], mbs[b], semr[b])
        return 0
    lax.fori_loop(0, nch // D, group, 0)
    for c in range(nch - nch % D, nch):
        b = c % D
        pltpu.make_async_copy(msg_hbm.at[rows(c)], mbs[b], semr[b]).wait()
        pltpu.sync_copy(mbs[b], acc.at[ivm.at[c]], add=True)
    plsc.subcore_barrier()

    @pl.when(sid < 15)
    def _():
        pltpu.sync_copy(acc.at[pl.ds(sid * NPTA, NPTA)],
                        part_hbm.at[cid, pl.ds(sid * NPTA, NPTA)])

    @pl.when(sid == 15)
    def _():
        pltpu.sync_copy(acc.at[pl.ds(15 * NPTA, NPTL)],
                        part_hbm.at[cid, pl.ds(15 * NPTA, NPTL)])


def _sc_scatter(msg, i2d, zrows, pt, nch):
    mesh = plsc.VectorSubcoreMesh(core_axis_name="c", subcore_axis_name="s")
    fn = pl.kernel(
        functools.partial(_scatter_body, pt, nch),
        out_type=jax.ShapeDtypeStruct((2, N_NODES, MROW), jnp.float32),
        mesh=mesh,
        scratch_types=[
            pltpu.VMEM((nch, CH), jnp.int32),
            pltpu.VMEM((CH, MROW), jnp.float32),
            pltpu.VMEM((CH, MROW), jnp.float32),
            pltpu.VMEM((CH, MROW), jnp.float32),
            pltpu.VMEM((CH, MROW), jnp.float32),
            pltpu.VMEM_SHARED((N_NODES, MROW), jnp.float32),
            pltpu.SemaphoreType.DMA,
            pltpu.SemaphoreType.DMA,
            pltpu.SemaphoreType.DMA,
            pltpu.SemaphoreType.DMA,
            pltpu.SemaphoreType.DMA,
            pltpu.SemaphoreType.DMA,
            pltpu.SemaphoreType.DMA,
            pltpu.SemaphoreType.DMA,
        ],
        compiler_params=_SC_PARAMS,
    )
    return fn(msg, i2d, zrows)


def _finish_body(pa, pb, wbT, m0, m1cm, m2cm):
    s = (pa[0] + pa[1] + pb[0] + pb[1]) * (1.0 / (NUM_AVG_NEIGH ** 0.5))
    outT = jnp.dot(wbT[...], s.T, preferred_element_type=jnp.float32)
    m0[...] = outT[0:8, :]
    m1cm[...] = outT[8:32, :]
    m2cm[...] = outT[32:104, :]


def _tc_finish(part_a, part_b, wbT):
    return pl.pallas_call(
        _finish_body,
        out_shape=(jax.ShapeDtypeStruct((8, N_NODES), jnp.float32),
                   jax.ShapeDtypeStruct((24, N_NODES), jnp.float32),
                   jax.ShapeDtypeStruct((72, N_NODES), jnp.float32)),
    )(part_a, part_b, wbT)


def kernel(edge_vector, edge_idx, atom_type, atom_feats_0, radial_W,
           mlp0_W0, mlp0_b0, mlp0_W1, mlp0_b1, mlp0_W2, mlp0_b2,
           mlp1_W0, mlp1_b0, mlp1_W1, mlp1_b1, mlp1_W2, mlp1_b2,
           mlp2_W0, mlp2_b0, mlp2_W1, mlp2_b1, mlp2_W2, mlp2_b2,
           chan0_W, chan1_W, chan2_W):
    f32 = jnp.float32
    i_row = edge_idx[0]
    j_row = edge_idx[1]
    ftab_flat = atom_feats_0.T.reshape(N_NODES * 8)

    tab = radial_W.reshape(16, N_U, N_CHEB).reshape(128, N_CHEB)
    w0 = jnp.concatenate([mlp0_W0, mlp1_W0, mlp2_W0], axis=1).T

    def bd(a, b, c):
        z = jnp.zeros((24, 24), f32)
        return z.at[0:8, 0:8].set(a).at[8:16, 8:16].set(b).at[16:24, 16:24].set(c)

    w1 = bd(mlp0_W1, mlp1_W1, mlp2_W1).T
    w2 = bd(mlp0_W2, mlp1_W2, mlp2_W2).T
    b0 = jnp.concatenate([mlp0_b0, mlp1_b0, mlp2_b0])[:, None]
    b1 = jnp.concatenate([mlp0_b1, mlp1_b1, mlp2_b1])[:, None]
    b2 = jnp.concatenate([mlp0_b2, mlp1_b2, mlp2_b2])[:, None]
    wb = jnp.zeros((MROW, MROW), f32)
    wb = wb.at[0:8, 0:8].set(chan0_W.T)
    wb = wb.at[8:32, 8:32].set(jnp.kron(jnp.eye(3, dtype=f32), chan1_W.T))
    wb = wb.at[32:104, 32:104].set(jnp.kron(jnp.eye(9, dtype=f32), chan2_W.T))

    zrows = jnp.zeros((N_NODES, MROW), f32)
    parts = []
    for (lo, half, pt, chunks) in ((0, HALF_A, PT_A, CHUNKS_A),
                                   (HALF_A, HALF_B, PT_B, CHUNKS_B)):
        nch = pt // CH
        ih = lax.slice_in_dim(i_row, lo, lo + half)
        jh = lax.slice_in_dim(j_row, lo, lo + half)
        evh = jnp.concatenate(
            [lax.slice_in_dim(edge_vector, lo, lo + half).T,
             jnp.zeros((5, half), f32)], axis=0)
        h0T, pair8 = _sc_gather(ih, jh, atom_type, ftab_flat, pt, chunks, half)
        msg = _tc_dense(evh, pair8, h0T, tab, w0, w1, w2, b0, b1, b2, half)
        i2d = ih.reshape(NW, nch, CH)
        parts.append(_sc_scatter(msg, i2d, zrows, pt, nch))

    M0, m1cm, m2cm = _tc_finish(parts[0], parts[1], wb.T)

    M1 = m1cm.reshape(3, 8, N_NODES).transpose(1, 2, 0)
    M2 = m2cm.reshape(9, 8, N_NODES).transpose(1, 2, 0).reshape(
        8, N_NODES, 3, 3)
    return (M0, M1, M2)
